# Initial kernel scaffold; baseline (speedup 1.0000x reference)
import jax
import jax.numpy as jnp
from jax.experimental import pallas as pl


def kernel(x, edge_index, pred_embed, W1l, b1l, W1r, W2l, b2l, W2r, Wm1, bm1, Wm2, bm2):
    raise NotImplementedError("write your pallas kernel here")


# speedup vs baseline: 2.2730x; 2.2730x over previous
import functools

import jax
import jax.numpy as jnp
from jax import lax
from jax.experimental import pallas as pl
from jax.experimental.pallas import tpu as pltpu
from jax.experimental.pallas import tpu_sc as plsc

N = 50000
E = 800000
NP = 50176
EP = 802816
CHUNK = 128
NWORK = 32
EDGES_PER_W = EP // NWORK
NCHUNK_E = EDGES_PER_W // CHUNK
NODE_CHUNKS = NP // CHUNK
ROWS_PER_TILE = NP // 16

_f32 = jnp.float32
_i32 = jnp.int32


def _vmesh():
    return plsc.VectorSubcoreMesh(
        core_axis_name="c", subcore_axis_name="s", num_cores=2, num_subcores=16
    )


_SC_PARAMS = pltpu.CompilerParams(
    needs_layout_passes=False, use_tc_tiling_on_sc=False
)


def _worker_id():
    return lax.axis_index("c") * 16 + lax.axis_index("s")


def _build_h0_body(xp_hbm, pred_hbm, h0_hbm, xv, pv, idxv, hv):
    w = _worker_id()
    iot = lax.iota(_i32, 16)

    @pl.loop(0, (NODE_CHUNKS + NWORK - 1) // NWORK)
    def _(i):
        c = w + i * NWORK

        @pl.when(c < NODE_CHUNKS)
        def _():
            base = c * CHUNK
            pltpu.sync_copy(xp_hbm.at[pl.ds(base, CHUNK)], xv)
            for g in range(8):
                rows = g * 16 + iot
                pidf = plsc.load_gather(xv, [rows, jnp.full((16,), 1, _i32)])
                plsc.store_scatter(idxv, [rows], pidf.astype(_i32))
            pltpu.sync_copy(pred_hbm.at[idxv], pv)
            for g in range(8):
                rows = g * 16 + iot
                v = plsc.load_gather(xv, [rows, jnp.full((16,), 0, _i32)])
                plsc.store_scatter(hv, [rows, jnp.full((16,), 0, _i32)], v)
                for j in range(8):
                    v = plsc.load_gather(xv, [rows, jnp.full((16,), 2 + j, _i32)])
                    plsc.store_scatter(hv, [rows, jnp.full((16,), 1 + j, _i32)], v)
                for j in range(16):
                    v = plsc.load_gather(pv, [rows, jnp.full((16,), j, _i32)])
                    plsc.store_scatter(hv, [rows, jnp.full((16,), 9 + j, _i32)], v)
                plsc.store_scatter(
                    hv, [rows, jnp.full((16,), 25, _i32)], jnp.ones((16,), _f32)
                )
                for j in range(26, 32):
                    plsc.store_scatter(
                        hv, [rows, jnp.full((16,), j, _i32)], jnp.zeros((16,), _f32)
                    )
            pltpu.sync_copy(hv, h0_hbm.at[pl.ds(base, CHUNK)])


def _build_h0(xp, pred):
    k = pl.kernel(
        _build_h0_body,
        out_type=jax.ShapeDtypeStruct((NP, 32), _f32),
        mesh=_vmesh(),
        compiler_params=_SC_PARAMS,
        scratch_types=[
            pltpu.VMEM((CHUNK, 16), _f32),
            pltpu.VMEM((CHUNK, 16), _f32),
            pltpu.VMEM((CHUNK,), _i32),
            pltpu.VMEM((CHUNK, 32), _f32),
        ],
    )
    return k(xp, pred)


def _segsum_body(table_hbm, src_hbm, dst_hbm, out_hbm, sidx, didx, rows_v, zbuf, acc):
    cid = lax.axis_index("c")
    sid = lax.axis_index("s")
    w = cid * 16 + sid

    @pl.loop(0, 64)
    def _(r):
        zbuf[r, pl.ds(0, 16)] = jnp.zeros((16,), _f32)
        zbuf[r, pl.ds(16, 16)] = jnp.zeros((16,), _f32)

    @pl.loop(0, ROWS_PER_TILE // 64)
    def _(j):
        pltpu.sync_copy(zbuf, acc.at[pl.ds(sid * ROWS_PER_TILE + j * 64, 64)])

    plsc.subcore_barrier()

    @pl.loop(0, NCHUNK_E)
    def _(c):
        base = w * EDGES_PER_W + c * CHUNK
        pltpu.sync_copy(src_hbm.at[pl.ds(base, CHUNK)], sidx)
        pltpu.sync_copy(dst_hbm.at[pl.ds(base, CHUNK)], didx)
        pltpu.sync_copy(table_hbm.at[sidx], rows_v)
        pltpu.sync_copy(rows_v, acc.at[didx], add=True)

    plsc.subcore_barrier()
    pltpu.sync_copy(
        acc.at[pl.ds(sid * ROWS_PER_TILE, ROWS_PER_TILE)],
        out_hbm.at[cid, pl.ds(sid * ROWS_PER_TILE, ROWS_PER_TILE)],
    )


def _segsum(table, srcp, dstp):
    k = pl.kernel(
        _segsum_body,
        out_type=jax.ShapeDtypeStruct((2, NP, 32), _f32),
        mesh=_vmesh(),
        compiler_params=_SC_PARAMS,
        scratch_types=[
            pltpu.VMEM((CHUNK,), _i32),
            pltpu.VMEM((CHUNK,), _i32),
            pltpu.VMEM((CHUNK, 32), _f32),
            pltpu.VMEM((64, 32), _f32),
            pltpu.VMEM_SHARED((NP, 32), _f32),
        ],
    )
    return k(table, srcp, dstp)


def _edge_body(
    a_hbm, b_hbm, src_hbm, dst_hbm, wm2_hbm, bm2_hbm, out_hbm,
    sidx, didx, ga, gb, outv, wm2_v, bm2_v,
):
    w = _worker_id()
    iot = lax.iota(_i32, 16)
    pltpu.sync_copy(wm2_hbm, wm2_v)
    pltpu.sync_copy(bm2_hbm, bm2_v)
    w0 = [wm2_v[0, pl.ds(j * 16, 16)] for j in range(4)]
    w1 = [wm2_v[1, pl.ds(j * 16, 16)] for j in range(4)]
    bv = bm2_v[...]

    @pl.loop(0, NCHUNK_E)
    def _(c):
        base = w * EDGES_PER_W + c * CHUNK
        pltpu.sync_copy(src_hbm.at[pl.ds(base, CHUNK)], sidx)
        pltpu.sync_copy(dst_hbm.at[pl.ds(base, CHUNK)], didx)
        pltpu.sync_copy(a_hbm.at[sidx], ga)
        pltpu.sync_copy(b_hbm.at[didx], gb)

        @pl.loop(0, 8)
        def _(g):
            rows = g * 16 + iot
            acc0 = jnp.zeros((16,), _f32) + bv[0]
            acc1 = jnp.zeros((16,), _f32) + bv[1]
            for k in range(64):
                ck = jnp.full((16,), k, _i32)
                a = plsc.load_gather(ga, [rows, ck])
                b = plsc.load_gather(gb, [rows, ck])
                r = jnp.maximum(a + b, 0.0)
                acc0 = acc0 + r * w0[k // 16][k % 16]
                acc1 = acc1 + r * w1[k // 16][k % 16]
            plsc.store_scatter(outv, [rows, jnp.full((16,), 0, _i32)], acc0)
            plsc.store_scatter(outv, [rows, jnp.full((16,), 1, _i32)], acc1)

        pltpu.sync_copy(outv, out_hbm.at[pl.ds(base, CHUNK)])


def _edge_mlp(a, b, srcp, dstp, wm2, bm2):
    k = pl.kernel(
        _edge_body,
        out_type=jax.ShapeDtypeStruct((EP, 2), _f32),
        mesh=_vmesh(),
        compiler_params=_SC_PARAMS,
        scratch_types=[
            pltpu.VMEM((CHUNK,), _i32),
            pltpu.VMEM((CHUNK,), _i32),
            pltpu.VMEM((CHUNK, 64), _f32),
            pltpu.VMEM((CHUNK, 64), _f32),
            pltpu.VMEM((CHUNK, 2), _f32),
            pltpu.VMEM((2, 64), _f32),
            pltpu.VMEM((16,), _f32),
        ],
    )
    return k(a, b, srcp, dstp, wm2, bm2)


_BLK = 512
_GRID = NP // _BLK


def _dot(a, b):
    return lax.dot_general(
        a, b, (((1,), (0,)), ((), ())), precision=lax.Precision.HIGHEST
    )


def _tc1_body(acc_ref, h0_ref, w1l_ref, w1r_ref, b1_ref, h1a_ref, h1b_ref, rcp_ref):
    acc = acc_ref[0] + acc_ref[1]
    cnt = acc[:, 25:26]
    rcp = 1.0 / jnp.maximum(cnt, 1.0)
    mean = acc * rcp
    out = _dot(mean, w1l_ref[...]) + _dot(h0_ref[...], w1r_ref[...]) + b1_ref[...]
    norm = jnp.sqrt(jnp.sum(out * out, axis=1, keepdims=True))
    h = jnp.maximum(out / jnp.maximum(norm, 1e-12), 0.0)
    h1a_ref[...] = h[:, :32]
    h1b_ref[...] = h[:, 32:]
    rcp_ref[...] = jnp.broadcast_to(rcp, (_BLK, 8))


def _tc1(acc, h0, w1lT, w1rT, b1):
    return pl.pallas_call(
        _tc1_body,
        grid=(_GRID,),
        in_specs=[
            pl.BlockSpec((2, _BLK, 32), lambda i: (0, i, 0)),
            pl.BlockSpec((_BLK, 32), lambda i: (i, 0)),
            pl.BlockSpec((32, 64), lambda i: (0, 0)),
            pl.BlockSpec((32, 64), lambda i: (0, 0)),
            pl.BlockSpec((1, 64), lambda i: (0, 0)),
        ],
        out_specs=[
            pl.BlockSpec((_BLK, 32), lambda i: (i, 0)),
            pl.BlockSpec((_BLK, 32), lambda i: (i, 0)),
            pl.BlockSpec((_BLK, 8), lambda i: (i, 0)),
        ],
        out_shape=[
            jax.ShapeDtypeStruct((NP, 32), _f32),
            jax.ShapeDtypeStruct((NP, 32), _f32),
            jax.ShapeDtypeStruct((NP, 8), _f32),
        ],
    )(acc, h0, w1lT, w1rT, b1)


def _tc2_body(
    acca_ref, accb_ref, h1a_ref, h1b_ref, rcp_ref,
    w2l_ref, w2r_ref, b2_ref, wm1s_ref, wm1d_ref, bm1_ref,
    a_ref, b_ref,
):
    rcp = rcp_ref[:, 0:1]
    mean = jnp.concatenate(
        [(acca_ref[0] + acca_ref[1]) * rcp, (accb_ref[0] + accb_ref[1]) * rcp], axis=1
    )
    h1 = jnp.concatenate([h1a_ref[...], h1b_ref[...]], axis=1)
    out = _dot(mean, w2l_ref[...]) + _dot(h1, w2r_ref[...]) + b2_ref[...]
    norm = jnp.sqrt(jnp.sum(out * out, axis=1, keepdims=True))
    h2 = jnp.maximum(out / jnp.maximum(norm, 1e-12), 0.0)
    a_ref[...] = _dot(h2, wm1s_ref[...])
    b_ref[...] = _dot(h2, wm1d_ref[...]) + bm1_ref[...]


def _tc2(acca, accb, h1a, h1b, rcp8, w2lT, w2rT, b2, wm1sT, wm1dT, bm1):
    return pl.pallas_call(
        _tc2_body,
        grid=(_GRID,),
        in_specs=[
            pl.BlockSpec((2, _BLK, 32), lambda i: (0, i, 0)),
            pl.BlockSpec((2, _BLK, 32), lambda i: (0, i, 0)),
            pl.BlockSpec((_BLK, 32), lambda i: (i, 0)),
            pl.BlockSpec((_BLK, 32), lambda i: (i, 0)),
            pl.BlockSpec((_BLK, 8), lambda i: (i, 0)),
            pl.BlockSpec((64, 64), lambda i: (0, 0)),
            pl.BlockSpec((64, 64), lambda i: (0, 0)),
            pl.BlockSpec((1, 64), lambda i: (0, 0)),
            pl.BlockSpec((64, 64), lambda i: (0, 0)),
            pl.BlockSpec((64, 64), lambda i: (0, 0)),
            pl.BlockSpec((1, 64), lambda i: (0, 0)),
        ],
        out_specs=[
            pl.BlockSpec((_BLK, 64), lambda i: (i, 0)),
            pl.BlockSpec((_BLK, 64), lambda i: (i, 0)),
        ],
        out_shape=[
            jax.ShapeDtypeStruct((NP, 64), _f32),
            jax.ShapeDtypeStruct((NP, 64), _f32),
        ],
    )(acca, accb, h1a, h1b, rcp8, w2lT, w2rT, b2, wm1sT, wm1dT, bm1)


def kernel(x, edge_index, pred_embed, W1l, b1l, W1r, W2l, b2l, W2r, Wm1, bm1, Wm2, bm2):
    xp = jnp.zeros((NP, 16), _f32).at[:N, :10].set(x)
    pad = jnp.full((EP - E,), N, _i32)
    srcp = jnp.concatenate([edge_index[0], pad])
    dstp = jnp.concatenate([edge_index[1], pad])

    w1lT = jnp.zeros((32, 64), _f32).at[:25].set(W1l.T)
    w1rT = jnp.zeros((32, 64), _f32).at[:25].set(W1r.T)

    h0 = _build_h0(xp, pred_embed)
    acc1 = _segsum(h0, srcp, dstp)
    h1a, h1b, rcp8 = _tc1(acc1, h0, w1lT, w1rT, b1l.reshape(1, 64))
    acca = _segsum(h1a, srcp, dstp)
    accb = _segsum(h1b, srcp, dstp)
    a, b = _tc2(
        acca, accb, h1a, h1b, rcp8,
        W2l.T, W2r.T, b2l.reshape(1, 64),
        Wm1[:, :64].T, Wm1[:, 64:].T, bm1.reshape(1, 64),
    )
    outp = _edge_mlp(a, b, srcp, dstp, Wm2, jnp.zeros((16,), _f32).at[:2].set(bm2))
    return outp[:E]


# speedup vs baseline: 3.8728x; 1.7038x over previous
import functools

import jax
import jax.numpy as jnp
from jax import lax
from jax.experimental import pallas as pl
from jax.experimental.pallas import tpu as pltpu
from jax.experimental.pallas import tpu_sc as plsc

N = 50000
E = 800000
NP = 50176
EP = 819200
CHUNK = 128
NWORK = 32
EDGES_PER_W = EP // NWORK
NCHUNK_E = EDGES_PER_W // CHUNK
NODE_CHUNKS = NP // CHUNK
ROWS_PER_TILE = NP // 16

_f32 = jnp.float32
_i32 = jnp.int32


def _vmesh():
    return plsc.VectorSubcoreMesh(
        core_axis_name="c", subcore_axis_name="s", num_cores=2, num_subcores=16
    )


_SC_PARAMS = pltpu.CompilerParams(
    needs_layout_passes=False, use_tc_tiling_on_sc=False
)


def _worker_id():
    return lax.axis_index("c") * 16 + lax.axis_index("s")


def _build_h0_body(xp_hbm, pred_hbm, h0_hbm, xv, pv, idxv, hv):
    w = _worker_id()
    iot = lax.iota(_i32, 16)

    @pl.loop(0, (NODE_CHUNKS + NWORK - 1) // NWORK)
    def _(i):
        c = w + i * NWORK

        @pl.when(c < NODE_CHUNKS)
        def _():
            base = c * CHUNK
            pltpu.sync_copy(xp_hbm.at[pl.ds(base, CHUNK)], xv)
            for g in range(8):
                rows = g * 16 + iot
                pidf = plsc.load_gather(xv, [rows, jnp.full((16,), 1, _i32)])
                plsc.store_scatter(idxv, [rows], pidf.astype(_i32))
            pltpu.sync_copy(pred_hbm.at[idxv], pv)
            for g in range(8):
                rows = g * 16 + iot
                v = plsc.load_gather(xv, [rows, jnp.full((16,), 0, _i32)])
                plsc.store_scatter(hv, [rows, jnp.full((16,), 0, _i32)], v)
                for j in range(8):
                    v = plsc.load_gather(xv, [rows, jnp.full((16,), 2 + j, _i32)])
                    plsc.store_scatter(hv, [rows, jnp.full((16,), 1 + j, _i32)], v)
                for j in range(16):
                    v = plsc.load_gather(pv, [rows, jnp.full((16,), j, _i32)])
                    plsc.store_scatter(hv, [rows, jnp.full((16,), 9 + j, _i32)], v)
                plsc.store_scatter(
                    hv, [rows, jnp.full((16,), 25, _i32)], jnp.ones((16,), _f32)
                )
                for j in range(26, 32):
                    plsc.store_scatter(
                        hv, [rows, jnp.full((16,), j, _i32)], jnp.zeros((16,), _f32)
                    )
            pltpu.sync_copy(hv, h0_hbm.at[pl.ds(base, CHUNK)])


def _build_h0(xp, pred):
    k = pl.kernel(
        _build_h0_body,
        out_type=jax.ShapeDtypeStruct((NP, 32), _f32),
        mesh=_vmesh(),
        compiler_params=_SC_PARAMS,
        scratch_types=[
            pltpu.VMEM((CHUNK, 16), _f32),
            pltpu.VMEM((CHUNK, 16), _f32),
            pltpu.VMEM((CHUNK,), _i32),
            pltpu.VMEM((CHUNK, 32), _f32),
        ],
    )
    return k(xp, pred)


def _segsum_body(
    table_hbm, src2_hbm, dst2_hbm, out_hbm,
    si, di, r0, r1, r2, r3, zbuf, acc, isem_s, isem_d, gsem, ssem,
):
    cid = lax.axis_index("c")
    sid = lax.axis_index("s")
    w = cid * 16 + sid
    rows = [r0, r1, r2, r3]

    @pl.loop(0, 64)
    def _(r):
        zbuf[r, pl.ds(0, 16)] = jnp.zeros((16,), _f32)
        zbuf[r, pl.ds(16, 16)] = jnp.zeros((16,), _f32)

    @pl.loop(0, ROWS_PER_TILE // 64)
    def _(j):
        pltpu.sync_copy(zbuf, acc.at[pl.ds(sid * ROWS_PER_TILE + j * 64, 64)])

    plsc.subcore_barrier()

    wbase = w * NCHUNK_E

    def fire_idx(c, m8):
        pltpu.async_copy(src2_hbm.at[wbase + c], si.at[m8], isem_s.at[m8])
        pltpu.async_copy(dst2_hbm.at[wbase + c], di.at[m8], isem_d.at[m8])

    def wait_idx(m8):
        pltpu.make_async_copy(src2_hbm.at[0], si.at[m8], isem_s.at[m8]).wait()
        pltpu.make_async_copy(dst2_hbm.at[0], di.at[m8], isem_d.at[m8]).wait()

    def fire_gather(m8, m4):
        pltpu.async_copy(table_hbm.at[si.at[m8]], rows[m4], gsem.at[m4])

    def wait_gather(m4):
        pltpu.make_async_copy(table_hbm.at[pl.ds(0, CHUNK)], rows[m4], gsem.at[m4]).wait()

    def fire_scatter(m8, m4):
        pltpu.async_copy(rows[m4], acc.at[di.at[m8]], ssem.at[m4], add=True)

    def wait_scatter(m4):
        pltpu.make_async_copy(rows[m4], acc.at[pl.ds(0, CHUNK)], ssem.at[m4]).wait()

    fire_idx(0, 0)
    fire_idx(1, 1)
    fire_idx(2, 2)
    wait_idx(0)
    fire_gather(0, 0)
    wait_idx(1)
    fire_gather(1, 1)

    @pl.loop(0, NCHUNK_E, step=8)
    def _(cc):
        for b in range(8):
            c = cc + b

            @pl.when(c + 3 < NCHUNK_E)
            def _():
                fire_idx(c + 3, (b + 3) % 8)

            wait_gather(b % 4)
            fire_scatter(b, b % 4)

            @pl.when(c + 2 < NCHUNK_E)
            def _():
                wait_idx((b + 2) % 8)

                @pl.when(c >= 2)
                def _():
                    wait_scatter((b + 2) % 4)

                fire_gather((b + 2) % 8, (b + 2) % 4)

    for b in range(4):
        wait_scatter(b)

    plsc.subcore_barrier()
    pltpu.sync_copy(
        acc.at[pl.ds(sid * ROWS_PER_TILE, ROWS_PER_TILE)],
        out_hbm.at[cid, pl.ds(sid * ROWS_PER_TILE, ROWS_PER_TILE)],
    )


def _segsum(table, src2, dst2):
    k = pl.kernel(
        _segsum_body,
        out_type=jax.ShapeDtypeStruct((2, NP, 32), _f32),
        mesh=_vmesh(),
        compiler_params=_SC_PARAMS,
        scratch_types=[
            pltpu.VMEM((8, CHUNK), _i32),
            pltpu.VMEM((8, CHUNK), _i32),
            pltpu.VMEM((CHUNK, 32), _f32),
            pltpu.VMEM((CHUNK, 32), _f32),
            pltpu.VMEM((CHUNK, 32), _f32),
            pltpu.VMEM((CHUNK, 32), _f32),
            pltpu.VMEM((64, 32), _f32),
            pltpu.VMEM_SHARED((NP, 32), _f32),
            pltpu.SemaphoreType.DMA((8,)),
            pltpu.SemaphoreType.DMA((8,)),
            pltpu.SemaphoreType.DMA((4,)),
            pltpu.SemaphoreType.DMA((4,)),
        ],
    )
    return k(table, src2, dst2)


def _edge_body(
    a_hbm, b_hbm, src_hbm, dst_hbm, wm2_hbm, bm2_hbm, out_hbm,
    sidx_all, didx_all, ga0, ga1, gb0, gb1, ov0, ov1, wm2_v, bm2_v, wrot,
    ga_s0, ga_s1, gb_s0, gb_s1, o_s0, o_s1,
):
    w = _worker_id()
    iot = lax.iota(_i32, 16)
    pltpu.sync_copy(wm2_hbm, wm2_v)
    pltpu.sync_copy(bm2_hbm, bm2_v)
    bv = bm2_v[...]

    @pl.loop(0, 64)
    def _(k):
        kl = jnp.bitwise_and(k + iot, 63)
        wrot[0, k] = plsc.load_gather(wm2_v, [jnp.zeros((16,), _i32), kl])
        wrot[1, k] = plsc.load_gather(wm2_v, [jnp.ones((16,), _i32), kl])

    pltpu.sync_copy(src_hbm.at[pl.ds(w * EDGES_PER_W, EDGES_PER_W)], sidx_all)
    pltpu.sync_copy(dst_hbm.at[pl.ds(w * EDGES_PER_W, EDGES_PER_W)], didx_all)
    ga = [ga0, ga1]
    gb = [gb0, gb1]
    ov = [ov0, ov1]
    gasem = [ga_s0, ga_s1]
    gbsem = [gb_s0, gb_s1]
    osem = [o_s0, o_s1]
    rows_g = [g * 16 + iot for g in range(8)]

    def fire_gather(c, p):
        pltpu.async_copy(
            a_hbm.at[sidx_all.at[pl.ds(c * CHUNK, CHUNK)]], ga[p], gasem[p]
        )
        pltpu.async_copy(
            b_hbm.at[didx_all.at[pl.ds(c * CHUNK, CHUNK)]], gb[p], gbsem[p]
        )

    def wait_gather(p):
        pltpu.make_async_copy(a_hbm.at[pl.ds(0, CHUNK)], ga[p], gasem[p]).wait()
        pltpu.make_async_copy(b_hbm.at[pl.ds(0, CHUNK)], gb[p], gbsem[p]).wait()

    fire_gather(0, 0)
    fire_gather(1, 1)

    @pl.loop(0, NCHUNK_E, step=2)
    def _(cc):
        for b in range(2):
            c = cc + b
            p = b
            wait_gather(p)

            @pl.when(c >= 2)
            def _():
                pltpu.make_async_copy(ov[p], out_hbm.at[pl.ds(0, CHUNK)], osem[p]).wait()

            init = tuple(jnp.zeros((16,), _f32) + bv[0] for _ in range(8)) + tuple(
                jnp.zeros((16,), _f32) + bv[1] for _ in range(8)
            )

            @pl.loop(0, 64, init_carry=init)
            def accs(k, carry):
                kl = jnp.bitwise_and(k + iot, 63)
                w0v = wrot[0, k]
                w1v = wrot[1, k]
                out = []
                out1 = []
                for g in range(8):
                    a = plsc.load_gather(ga[p], [rows_g[g], kl])
                    bb = plsc.load_gather(gb[p], [rows_g[g], kl])
                    r = jnp.maximum(a + bb, 0.0)
                    out.append(carry[g] + r * w0v)
                    out1.append(carry[8 + g] + r * w1v)
                return tuple(out) + tuple(out1)

            for g in range(8):
                plsc.store_scatter(ov[p], [rows_g[g], jnp.full((16,), 0, _i32)], accs[g])
                plsc.store_scatter(
                    ov[p], [rows_g[g], jnp.full((16,), 1, _i32)], accs[8 + g]
                )

            base = w * EDGES_PER_W + c * CHUNK
            pltpu.async_copy(ov[p], out_hbm.at[pl.ds(base, CHUNK)], osem[p])

            @pl.when(c + 2 < NCHUNK_E)
            def _():
                fire_gather(c + 2, p)

    for b in range(2):
        pltpu.make_async_copy(ov[b], out_hbm.at[pl.ds(0, CHUNK)], osem[b]).wait()


def _edge_mlp(a, b, srcp, dstp, wm2, bm2):
    k = pl.kernel(
        _edge_body,
        out_type=jax.ShapeDtypeStruct((EP, 2), _f32),
        mesh=_vmesh(),
        compiler_params=_SC_PARAMS,
        scratch_types=[
            pltpu.VMEM((EDGES_PER_W,), _i32),
            pltpu.VMEM((EDGES_PER_W,), _i32),
            pltpu.VMEM((CHUNK, 64), _f32),
            pltpu.VMEM((CHUNK, 64), _f32),
            pltpu.VMEM((CHUNK, 64), _f32),
            pltpu.VMEM((CHUNK, 64), _f32),
            pltpu.VMEM((CHUNK, 2), _f32),
            pltpu.VMEM((CHUNK, 2), _f32),
            pltpu.VMEM((2, 64), _f32),
            pltpu.VMEM((16,), _f32),
            pltpu.VMEM((2, 64, 16), _f32),
        ] + [pltpu.SemaphoreType.DMA] * 6,
    )
    return k(a, b, srcp, dstp, wm2, bm2)


_BLK = 512
_GRID = NP // _BLK


def _dot(a, b):
    return lax.dot_general(
        a, b, (((1,), (0,)), ((), ())), precision=lax.Precision.HIGHEST
    )


def _tc1_body(acc_ref, h0_ref, w1l_ref, w1r_ref, b1_ref, h1a_ref, h1b_ref, rcp_ref):
    acc = acc_ref[0] + acc_ref[1]
    cnt = acc[:, 25:26]
    rcp = 1.0 / jnp.maximum(cnt, 1.0)
    mean = acc * rcp
    out = _dot(mean, w1l_ref[...]) + _dot(h0_ref[...], w1r_ref[...]) + b1_ref[...]
    norm = jnp.sqrt(jnp.sum(out * out, axis=1, keepdims=True))
    h = jnp.maximum(out / jnp.maximum(norm, 1e-12), 0.0)
    h1a_ref[...] = h[:, :32]
    h1b_ref[...] = h[:, 32:]
    rcp_ref[...] = jnp.broadcast_to(rcp, (_BLK, 8))


def _tc1(acc, h0, w1lT, w1rT, b1):
    return pl.pallas_call(
        _tc1_body,
        grid=(_GRID,),
        in_specs=[
            pl.BlockSpec((2, _BLK, 32), lambda i: (0, i, 0)),
            pl.BlockSpec((_BLK, 32), lambda i: (i, 0)),
            pl.BlockSpec((32, 64), lambda i: (0, 0)),
            pl.BlockSpec((32, 64), lambda i: (0, 0)),
            pl.BlockSpec((1, 64), lambda i: (0, 0)),
        ],
        out_specs=[
            pl.BlockSpec((_BLK, 32), lambda i: (i, 0)),
            pl.BlockSpec((_BLK, 32), lambda i: (i, 0)),
            pl.BlockSpec((_BLK, 8), lambda i: (i, 0)),
        ],
        out_shape=[
            jax.ShapeDtypeStruct((NP, 32), _f32),
            jax.ShapeDtypeStruct((NP, 32), _f32),
            jax.ShapeDtypeStruct((NP, 8), _f32),
        ],
    )(acc, h0, w1lT, w1rT, b1)


def _tc2_body(
    acca_ref, accb_ref, h1a_ref, h1b_ref, rcp_ref,
    w2l_ref, w2r_ref, b2_ref, wm1s_ref, wm1d_ref, bm1_ref,
    a_ref, b_ref,
):
    rcp = rcp_ref[:, 0:1]
    mean = jnp.concatenate(
        [(acca_ref[0] + acca_ref[1]) * rcp, (accb_ref[0] + accb_ref[1]) * rcp], axis=1
    )
    h1 = jnp.concatenate([h1a_ref[...], h1b_ref[...]], axis=1)
    out = _dot(mean, w2l_ref[...]) + _dot(h1, w2r_ref[...]) + b2_ref[...]
    norm = jnp.sqrt(jnp.sum(out * out, axis=1, keepdims=True))
    h2 = jnp.maximum(out / jnp.maximum(norm, 1e-12), 0.0)
    a_ref[...] = _dot(h2, wm1s_ref[...])
    b_ref[...] = _dot(h2, wm1d_ref[...]) + bm1_ref[...]


def _tc2(acca, accb, h1a, h1b, rcp8, w2lT, w2rT, b2, wm1sT, wm1dT, bm1):
    return pl.pallas_call(
        _tc2_body,
        grid=(_GRID,),
        in_specs=[
            pl.BlockSpec((2, _BLK, 32), lambda i: (0, i, 0)),
            pl.BlockSpec((2, _BLK, 32), lambda i: (0, i, 0)),
            pl.BlockSpec((_BLK, 32), lambda i: (i, 0)),
            pl.BlockSpec((_BLK, 32), lambda i: (i, 0)),
            pl.BlockSpec((_BLK, 8), lambda i: (i, 0)),
            pl.BlockSpec((64, 64), lambda i: (0, 0)),
            pl.BlockSpec((64, 64), lambda i: (0, 0)),
            pl.BlockSpec((1, 64), lambda i: (0, 0)),
            pl.BlockSpec((64, 64), lambda i: (0, 0)),
            pl.BlockSpec((64, 64), lambda i: (0, 0)),
            pl.BlockSpec((1, 64), lambda i: (0, 0)),
        ],
        out_specs=[
            pl.BlockSpec((_BLK, 64), lambda i: (i, 0)),
            pl.BlockSpec((_BLK, 64), lambda i: (i, 0)),
        ],
        out_shape=[
            jax.ShapeDtypeStruct((NP, 64), _f32),
            jax.ShapeDtypeStruct((NP, 64), _f32),
        ],
    )(acca, accb, h1a, h1b, rcp8, w2lT, w2rT, b2, wm1sT, wm1dT, bm1)


def kernel(x, edge_index, pred_embed, W1l, b1l, W1r, W2l, b2l, W2r, Wm1, bm1, Wm2, bm2):
    xp = jnp.zeros((NP, 16), _f32).at[:N, :10].set(x)
    pad = jnp.full((EP - E,), N, _i32)
    srcp = jnp.concatenate([edge_index[0], pad])
    dstp = jnp.concatenate([edge_index[1], pad])

    w1lT = jnp.zeros((32, 64), _f32).at[:25].set(W1l.T)
    w1rT = jnp.zeros((32, 64), _f32).at[:25].set(W1r.T)

    src2 = srcp.reshape(EP // CHUNK, CHUNK)
    dst2 = dstp.reshape(EP // CHUNK, CHUNK)

    h0 = _build_h0(xp, pred_embed)
    acc1 = _segsum(h0, src2, dst2)
    h1a, h1b, rcp8 = _tc1(acc1, h0, w1lT, w1rT, b1l.reshape(1, 64))
    acca = _segsum(h1a, src2, dst2)
    accb = _segsum(h1b, src2, dst2)
    a, b = _tc2(
        acca, accb, h1a, h1b, rcp8,
        W2l.T, W2r.T, b2l.reshape(1, 64),
        Wm1[:, :64].T, Wm1[:, 64:].T, bm1.reshape(1, 64),
    )
    outp = _edge_mlp(a, b, srcp, dstp, Wm2, jnp.zeros((16,), _f32).at[:2].set(bm2))
    return outp[:E]


# speedup vs baseline: 5.2347x; 1.3517x over previous
import functools

import jax
import jax.numpy as jnp
from jax import lax
from jax.experimental import pallas as pl
from jax.experimental.pallas import tpu as pltpu
from jax.experimental.pallas import tpu_sc as plsc

N = 50000
E = 800000
NP = 50176
EP = 819200
CHUNK = 128
NWORK = 32
EDGES_PER_W = EP // NWORK
NCHUNK_E = EDGES_PER_W // CHUNK
NC_EDGE = (296, 104)
NC_SEG = (264, 136)
NCMAX_EDGE = max(NC_EDGE)
IDX_PAD = NCMAX_EDGE * CHUNK
EPI = (16 * NC_EDGE[0] + 15 * NC_EDGE[1]) * CHUNK + IDX_PAD
NODE_CHUNKS = NP // CHUNK
ROWS_PER_TILE = NP // 16

_f32 = jnp.float32
_i32 = jnp.int32


def _vmesh():
    return plsc.VectorSubcoreMesh(
        core_axis_name="c", subcore_axis_name="s", num_cores=2, num_subcores=16
    )


_SC_PARAMS = pltpu.CompilerParams(
    needs_layout_passes=False, use_tc_tiling_on_sc=False
)


def _worker_id():
    return lax.axis_index("c") * 16 + lax.axis_index("s")


def _build_h0_body(xp_hbm, pred_hbm, h0_hbm, xv, pv, idxv, hv):
    w = _worker_id()
    iot = lax.iota(_i32, 16)

    @pl.loop(0, (NODE_CHUNKS + NWORK - 1) // NWORK)
    def _(i):
        c = w + i * NWORK

        @pl.when(c < NODE_CHUNKS)
        def _():
            base = c * CHUNK
            pltpu.sync_copy(xp_hbm.at[pl.ds(base, CHUNK)], xv)
            for g in range(8):
                rows = g * 16 + iot
                pidf = plsc.load_gather(xv, [rows, jnp.full((16,), 1, _i32)])
                plsc.store_scatter(idxv, [rows], pidf.astype(_i32))
            pltpu.sync_copy(pred_hbm.at[idxv], pv)
            for g in range(8):
                rows = g * 16 + iot
                v = plsc.load_gather(xv, [rows, jnp.full((16,), 0, _i32)])
                plsc.store_scatter(hv, [rows, jnp.full((16,), 0, _i32)], v)
                for j in range(8):
                    v = plsc.load_gather(xv, [rows, jnp.full((16,), 2 + j, _i32)])
                    plsc.store_scatter(hv, [rows, jnp.full((16,), 1 + j, _i32)], v)
                for j in range(16):
                    v = plsc.load_gather(pv, [rows, jnp.full((16,), j, _i32)])
                    plsc.store_scatter(hv, [rows, jnp.full((16,), 9 + j, _i32)], v)
                plsc.store_scatter(
                    hv, [rows, jnp.full((16,), 25, _i32)], jnp.ones((16,), _f32)
                )
                for j in range(26, 32):
                    plsc.store_scatter(
                        hv, [rows, jnp.full((16,), j, _i32)], jnp.zeros((16,), _f32)
                    )
            pltpu.sync_copy(hv, h0_hbm.at[pl.ds(base, CHUNK)])


def _build_h0(xp, pred):
    k = pl.kernel(
        _build_h0_body,
        out_type=jax.ShapeDtypeStruct((NP, 32), _f32),
        mesh=_vmesh(),
        compiler_params=_SC_PARAMS,
        scratch_types=[
            pltpu.VMEM((CHUNK, 16), _f32),
            pltpu.VMEM((CHUNK, 16), _f32),
            pltpu.VMEM((CHUNK,), _i32),
            pltpu.VMEM((CHUNK, 32), _f32),
        ],
    )
    return k(xp, pred)


def _segsum_body(
    table_hbm, src2_hbm, dst2_hbm, out_hbm,
    si, di, r0, r1, r2, r3, zbuf, acc, isem_s, isem_d, gsem, ssem,
):
    cid = lax.axis_index("c")
    sid = lax.axis_index("s")
    nc = jnp.where(cid == 0, NC_SEG[0], NC_SEG[1])
    wbase = jnp.where(cid == 0, sid * NC_SEG[0], 16 * NC_SEG[0] + sid * NC_SEG[1])
    rows = [r0, r1, r2, r3]

    @pl.loop(0, 64)
    def _(r):
        zbuf[r, pl.ds(0, 16)] = jnp.zeros((16,), _f32)
        zbuf[r, pl.ds(16, 16)] = jnp.zeros((16,), _f32)

    @pl.loop(0, ROWS_PER_TILE // 64)
    def _(j):
        pltpu.sync_copy(zbuf, acc.at[pl.ds(sid * ROWS_PER_TILE + j * 64, 64)])

    plsc.subcore_barrier()

    def fire_idx(c, m8):
        pltpu.async_copy(src2_hbm.at[wbase + c], si.at[m8], isem_s.at[m8])
        pltpu.async_copy(dst2_hbm.at[wbase + c], di.at[m8], isem_d.at[m8])

    def wait_idx(m8):
        pltpu.make_async_copy(src2_hbm.at[0], si.at[m8], isem_s.at[m8]).wait()
        pltpu.make_async_copy(dst2_hbm.at[0], di.at[m8], isem_d.at[m8]).wait()

    def fire_gather(m8, m4):
        pltpu.async_copy(table_hbm.at[si.at[m8]], rows[m4], gsem.at[m4])

    def wait_gather(m4):
        pltpu.make_async_copy(table_hbm.at[pl.ds(0, CHUNK)], rows[m4], gsem.at[m4]).wait()

    def fire_scatter(m8, m4):
        pltpu.async_copy(rows[m4], acc.at[di.at[m8]], ssem.at[m4], add=True)

    def wait_scatter(m4):
        pltpu.make_async_copy(rows[m4], acc.at[pl.ds(0, CHUNK)], ssem.at[m4]).wait()

    fire_idx(0, 0)
    fire_idx(1, 1)
    fire_idx(2, 2)
    wait_idx(0)
    fire_gather(0, 0)
    wait_idx(1)
    fire_gather(1, 1)

    @pl.loop(0, nc, step=8)
    def _(cc):
        for b in range(8):
            c = cc + b

            @pl.when(c + 3 < nc)
            def _():
                fire_idx(c + 3, (b + 3) % 8)

            wait_gather(b % 4)
            fire_scatter(b, b % 4)

            @pl.when(c + 2 < nc)
            def _():
                wait_idx((b + 2) % 8)

                @pl.when(c >= 2)
                def _():
                    wait_scatter((b + 2) % 4)

                fire_gather((b + 2) % 8, (b + 2) % 4)

    for b in range(4):
        wait_scatter(b)

    plsc.subcore_barrier()
    pltpu.sync_copy(
        acc.at[pl.ds(sid * ROWS_PER_TILE, ROWS_PER_TILE)],
        out_hbm.at[cid, pl.ds(sid * ROWS_PER_TILE, ROWS_PER_TILE)],
    )


def _segsum(table, src2, dst2):
    k = pl.kernel(
        _segsum_body,
        out_type=jax.ShapeDtypeStruct((2, NP, 32), _f32),
        mesh=_vmesh(),
        compiler_params=_SC_PARAMS,
        scratch_types=[
            pltpu.VMEM((8, CHUNK), _i32),
            pltpu.VMEM((8, CHUNK), _i32),
            pltpu.VMEM((CHUNK, 32), _f32),
            pltpu.VMEM((CHUNK, 32), _f32),
            pltpu.VMEM((CHUNK, 32), _f32),
            pltpu.VMEM((CHUNK, 32), _f32),
            pltpu.VMEM((64, 32), _f32),
            pltpu.VMEM_SHARED((NP, 32), _f32),
            pltpu.SemaphoreType.DMA((8,)),
            pltpu.SemaphoreType.DMA((8,)),
            pltpu.SemaphoreType.DMA((4,)),
            pltpu.SemaphoreType.DMA((4,)),
        ],
    )
    return k(table, src2, dst2)


def _edge_body(
    a_hbm, b_hbm, src_hbm, dst_hbm, wm2_hbm, bm2_hbm, o0_hbm, o1_hbm,
    sidx_all, didx_all, ga0, ga1, gb0, gb1, ov00, ov01, ov10, ov11, wm2_v, bm2_v, wrot,
    ga_s0, ga_s1, gb_s0, gb_s1, o0_s0, o0_s1, o1_s0, o1_s1,
):
    cid = lax.axis_index("c")
    sid = lax.axis_index("s")
    nc = jnp.where(cid == 0, NC_EDGE[0], NC_EDGE[1])
    wstart = jnp.where(cid == 0, sid * NC_EDGE[0], 16 * NC_EDGE[0] + sid * NC_EDGE[1])
    ebase = wstart * CHUNK
    iot = lax.iota(_i32, 16)
    pltpu.sync_copy(wm2_hbm, wm2_v)
    pltpu.sync_copy(bm2_hbm, bm2_v)
    bv = bm2_v[...]

    @pl.loop(0, 64)
    def _(k):
        kl = jnp.bitwise_and(k + iot, 63)
        wrot[0, k] = plsc.load_gather(wm2_v, [jnp.zeros((16,), _i32), kl])
        wrot[1, k] = plsc.load_gather(wm2_v, [jnp.ones((16,), _i32), kl])

    pltpu.sync_copy(src_hbm.at[pl.ds(ebase, IDX_PAD)], sidx_all)
    pltpu.sync_copy(dst_hbm.at[pl.ds(ebase, IDX_PAD)], didx_all)
    ga = [ga0, ga1]
    gb = [gb0, gb1]
    ov0 = [ov00, ov01]
    ov1 = [ov10, ov11]
    gasem = [ga_s0, ga_s1]
    gbsem = [gb_s0, gb_s1]
    o0sem = [o0_s0, o0_s1]
    o1sem = [o1_s0, o1_s1]
    rows_g = [g * 16 + iot for g in range(8)]

    def fire_gather(c, p):
        pltpu.async_copy(
            a_hbm.at[sidx_all.at[pl.ds(c * CHUNK, CHUNK)]], ga[p], gasem[p]
        )
        pltpu.async_copy(
            b_hbm.at[didx_all.at[pl.ds(c * CHUNK, CHUNK)]], gb[p], gbsem[p]
        )

    def wait_gather(p):
        pltpu.make_async_copy(a_hbm.at[pl.ds(0, CHUNK)], ga[p], gasem[p]).wait()
        pltpu.make_async_copy(b_hbm.at[pl.ds(0, CHUNK)], gb[p], gbsem[p]).wait()

    def wait_out(p):
        pltpu.make_async_copy(ov0[p], o0_hbm.at[pl.ds(0, CHUNK)], o0sem[p]).wait()
        pltpu.make_async_copy(ov1[p], o1_hbm.at[pl.ds(0, CHUNK)], o1sem[p]).wait()

    fire_gather(0, 0)
    fire_gather(1, 1)

    @pl.loop(0, nc, step=2)
    def _(cc):
        for b in range(2):
            c = cc + b
            p = b
            wait_gather(p)

            @pl.when(c >= 2)
            def _():
                wait_out(p)

            init = tuple(jnp.zeros((16,), _f32) + bv[0] for _ in range(8)) + tuple(
                jnp.zeros((16,), _f32) + bv[1] for _ in range(8)
            )

            @pl.loop(0, 64, init_carry=init)
            def accs(k, carry):
                kl = jnp.bitwise_and(k + iot, 63)
                w0v = wrot[0, k]
                w1v = wrot[1, k]
                out = []
                out1 = []
                for g in range(8):
                    a = plsc.load_gather(ga[p], [rows_g[g], kl])
                    bb = plsc.load_gather(gb[p], [rows_g[g], kl])
                    r = jnp.maximum(a + bb, 0.0)
                    out.append(carry[g] + r * w0v)
                    out1.append(carry[8 + g] + r * w1v)
                return tuple(out) + tuple(out1)

            for g in range(8):
                ov0[p][pl.ds(g * 16, 16)] = accs[g]
                ov1[p][pl.ds(g * 16, 16)] = accs[8 + g]

            base = ebase + c * CHUNK
            pltpu.async_copy(ov0[p], o0_hbm.at[pl.ds(base, CHUNK)], o0sem[p])
            pltpu.async_copy(ov1[p], o1_hbm.at[pl.ds(base, CHUNK)], o1sem[p])

            @pl.when(c + 2 < nc)
            def _():
                fire_gather(c + 2, p)

    for b in range(2):
        wait_out(b)


def _edge_mlp(a, b, srcp, dstp, wm2, bm2):
    k = pl.kernel(
        _edge_body,
        out_type=(
            jax.ShapeDtypeStruct((EP,), _f32),
            jax.ShapeDtypeStruct((EP,), _f32),
        ),
        mesh=_vmesh(),
        compiler_params=_SC_PARAMS,
        scratch_types=[
            pltpu.VMEM((IDX_PAD,), _i32),
            pltpu.VMEM((IDX_PAD,), _i32),
            pltpu.VMEM((CHUNK, 64), _f32),
            pltpu.VMEM((CHUNK, 64), _f32),
            pltpu.VMEM((CHUNK, 64), _f32),
            pltpu.VMEM((CHUNK, 64), _f32),
            pltpu.VMEM((CHUNK,), _f32),
            pltpu.VMEM((CHUNK,), _f32),
            pltpu.VMEM((CHUNK,), _f32),
            pltpu.VMEM((CHUNK,), _f32),
            pltpu.VMEM((2, 64), _f32),
            pltpu.VMEM((16,), _f32),
            pltpu.VMEM((2, 64, 16), _f32),
        ] + [pltpu.SemaphoreType.DMA] * 8,
    )
    return k(a, b, srcp, dstp, wm2, bm2)


_BLK = 1024
_GRID = NP // _BLK


def _dot(a, b):
    return lax.dot_general(
        a, b, (((1,), (0,)), ((), ())), precision=lax.Precision.HIGHEST
    )


def _tc1_body(acc_ref, h0_ref, w1l_ref, w1r_ref, b1_ref, h1a_ref, h1b_ref, rcp_ref):
    acc = acc_ref[0] + acc_ref[1]
    cnt = acc[:, 25:26]
    rcp = 1.0 / jnp.maximum(cnt, 1.0)
    mean = acc * rcp
    out = _dot(mean, w1l_ref[...]) + _dot(h0_ref[...], w1r_ref[...]) + b1_ref[...]
    norm = jnp.sqrt(jnp.sum(out * out, axis=1, keepdims=True))
    h = jnp.maximum(out / jnp.maximum(norm, 1e-12), 0.0)
    h1a_ref[...] = h[:, :32]
    h1b_ref[...] = h[:, 32:]
    rcp_ref[...] = jnp.broadcast_to(rcp, (_BLK, 8))


def _tc1(acc, h0, w1lT, w1rT, b1):
    return pl.pallas_call(
        _tc1_body,
        grid=(_GRID,),
        in_specs=[
            pl.BlockSpec((2, _BLK, 32), lambda i: (0, i, 0)),
            pl.BlockSpec((_BLK, 32), lambda i: (i, 0)),
            pl.BlockSpec((32, 64), lambda i: (0, 0)),
            pl.BlockSpec((32, 64), lambda i: (0, 0)),
            pl.BlockSpec((1, 64), lambda i: (0, 0)),
        ],
        out_specs=[
            pl.BlockSpec((_BLK, 32), lambda i: (i, 0)),
            pl.BlockSpec((_BLK, 32), lambda i: (i, 0)),
            pl.BlockSpec((_BLK, 8), lambda i: (i, 0)),
        ],
        out_shape=[
            jax.ShapeDtypeStruct((NP, 32), _f32),
            jax.ShapeDtypeStruct((NP, 32), _f32),
            jax.ShapeDtypeStruct((NP, 8), _f32),
        ],
    )(acc, h0, w1lT, w1rT, b1)


def _tc2_body(
    acca_ref, accb_ref, h1a_ref, h1b_ref, rcp_ref,
    w2l_ref, w2r_ref, b2_ref, wm1s_ref, wm1d_ref, bm1_ref,
    a_ref, b_ref,
):
    rcp = rcp_ref[:, 0:1]
    mean = jnp.concatenate(
        [(acca_ref[0] + acca_ref[1]) * rcp, (accb_ref[0] + accb_ref[1]) * rcp], axis=1
    )
    h1 = jnp.concatenate([h1a_ref[...], h1b_ref[...]], axis=1)
    out = _dot(mean, w2l_ref[...]) + _dot(h1, w2r_ref[...]) + b2_ref[...]
    norm = jnp.sqrt(jnp.sum(out * out, axis=1, keepdims=True))
    h2 = jnp.maximum(out / jnp.maximum(norm, 1e-12), 0.0)
    a_ref[...] = _dot(h2, wm1s_ref[...])
    b_ref[...] = _dot(h2, wm1d_ref[...]) + bm1_ref[...]


def _tc2(acca, accb, h1a, h1b, rcp8, w2lT, w2rT, b2, wm1sT, wm1dT, bm1):
    return pl.pallas_call(
        _tc2_body,
        grid=(_GRID,),
        in_specs=[
            pl.BlockSpec((2, _BLK, 32), lambda i: (0, i, 0)),
            pl.BlockSpec((2, _BLK, 32), lambda i: (0, i, 0)),
            pl.BlockSpec((_BLK, 32), lambda i: (i, 0)),
            pl.BlockSpec((_BLK, 32), lambda i: (i, 0)),
            pl.BlockSpec((_BLK, 8), lambda i: (i, 0)),
            pl.BlockSpec((64, 64), lambda i: (0, 0)),
            pl.BlockSpec((64, 64), lambda i: (0, 0)),
            pl.BlockSpec((1, 64), lambda i: (0, 0)),
            pl.BlockSpec((64, 64), lambda i: (0, 0)),
            pl.BlockSpec((64, 64), lambda i: (0, 0)),
            pl.BlockSpec((1, 64), lambda i: (0, 0)),
        ],
        out_specs=[
            pl.BlockSpec((_BLK, 64), lambda i: (i, 0)),
            pl.BlockSpec((_BLK, 64), lambda i: (i, 0)),
        ],
        out_shape=[
            jax.ShapeDtypeStruct((NP, 64), _f32),
            jax.ShapeDtypeStruct((NP, 64), _f32),
        ],
    )(acca, accb, h1a, h1b, rcp8, w2lT, w2rT, b2, wm1sT, wm1dT, bm1)


def kernel(x, edge_index, pred_embed, W1l, b1l, W1r, W2l, b2l, W2r, Wm1, bm1, Wm2, bm2):
    xp = jnp.zeros((NP, 16), _f32).at[:N, :10].set(x)
    pad = jnp.full((EPI - E,), N, _i32)
    srcp = jnp.concatenate([edge_index[0], pad])
    dstp = jnp.concatenate([edge_index[1], pad])

    w1lT = jnp.zeros((32, 64), _f32).at[:25].set(W1l.T)
    w1rT = jnp.zeros((32, 64), _f32).at[:25].set(W1r.T)

    src2 = srcp.reshape(EPI // CHUNK, CHUNK)
    dst2 = dstp.reshape(EPI // CHUNK, CHUNK)

    h0 = _build_h0(xp, pred_embed)
    acc1 = _segsum(h0, src2, dst2)
    h1a, h1b, rcp8 = _tc1(acc1, h0, w1lT, w1rT, b1l.reshape(1, 64))
    acca = _segsum(h1a, src2, dst2)
    accb = _segsum(h1b, src2, dst2)
    a, b = _tc2(
        acca, accb, h1a, h1b, rcp8,
        W2l.T, W2r.T, b2l.reshape(1, 64),
        Wm1[:, :64].T, Wm1[:, 64:].T, bm1.reshape(1, 64),
    )
    o0, o1 = _edge_mlp(a, b, srcp, dstp, Wm2, jnp.zeros((16,), _f32).at[:2].set(bm2))
    return jnp.stack([o0[:E], o1[:E]], axis=1)


# speedup vs baseline: 5.8757x; 1.1225x over previous
import functools

import jax
import jax.numpy as jnp
from jax import lax
from jax.experimental import pallas as pl
from jax.experimental.pallas import tpu as pltpu
from jax.experimental.pallas import tpu_sc as plsc

N = 50000
E = 800000
NP = 50176
EP = 819200
CHUNK = 128
NWORK = 32
EDGES_PER_W = EP // NWORK
NCHUNK_E = EDGES_PER_W // CHUNK
NC_EDGE = (296, 104)
NC_SEG = (304, 96)
NCMAX_EDGE = max(NC_EDGE)
QS = 20000.0
IDX_PAD = NCMAX_EDGE * CHUNK
EPI = (16 * NC_EDGE[0] + 15 * NC_EDGE[1]) * CHUNK + IDX_PAD
NODE_CHUNKS = NP // CHUNK
ROWS_PER_TILE = NP // 16

_f32 = jnp.float32
_i32 = jnp.int32


def _vmesh():
    return plsc.VectorSubcoreMesh(
        core_axis_name="c", subcore_axis_name="s", num_cores=2, num_subcores=16
    )


_SC_PARAMS = pltpu.CompilerParams(
    needs_layout_passes=False, use_tc_tiling_on_sc=False
)


def _worker_id():
    return lax.axis_index("c") * 16 + lax.axis_index("s")


def _build_h0_body(xp_hbm, pred_hbm, h0_hbm, xv, pv, idxv, hv):
    w = _worker_id()
    iot = lax.iota(_i32, 16)

    @pl.loop(0, (NODE_CHUNKS + NWORK - 1) // NWORK)
    def _(i):
        c = w + i * NWORK

        @pl.when(c < NODE_CHUNKS)
        def _():
            base = c * CHUNK
            pltpu.sync_copy(xp_hbm.at[pl.ds(base, CHUNK)], xv)
            for g in range(8):
                rows = g * 16 + iot
                pidf = plsc.load_gather(xv, [rows, jnp.full((16,), 1, _i32)])
                plsc.store_scatter(idxv, [rows], pidf.astype(_i32))
            pltpu.sync_copy(pred_hbm.at[idxv], pv)
            for g in range(8):
                rows = g * 16 + iot
                v = plsc.load_gather(xv, [rows, jnp.full((16,), 0, _i32)])
                plsc.store_scatter(hv, [rows, jnp.full((16,), 0, _i32)], v)
                for j in range(8):
                    v = plsc.load_gather(xv, [rows, jnp.full((16,), 2 + j, _i32)])
                    plsc.store_scatter(hv, [rows, jnp.full((16,), 1 + j, _i32)], v)
                for j in range(16):
                    v = plsc.load_gather(pv, [rows, jnp.full((16,), j, _i32)])
                    plsc.store_scatter(hv, [rows, jnp.full((16,), 9 + j, _i32)], v)
                plsc.store_scatter(
                    hv, [rows, jnp.full((16,), 25, _i32)], jnp.ones((16,), _f32)
                )
                for j in range(26, 32):
                    plsc.store_scatter(
                        hv, [rows, jnp.full((16,), j, _i32)], jnp.zeros((16,), _f32)
                    )
            pltpu.sync_copy(hv, h0_hbm.at[pl.ds(base, CHUNK)])


def _build_h0(xp, pred):
    k = pl.kernel(
        _build_h0_body,
        out_type=jax.ShapeDtypeStruct((NP, 32), _f32),
        mesh=_vmesh(),
        compiler_params=_SC_PARAMS,
        scratch_types=[
            pltpu.VMEM((CHUNK, 16), _f32),
            pltpu.VMEM((CHUNK, 16), _f32),
            pltpu.VMEM((CHUNK,), _i32),
            pltpu.VMEM((CHUNK, 32), _f32),
        ],
    )
    return k(xp, pred)


def _segsum_body(
    table_hbm, src2_hbm, dst2_hbm, out_hbm,
    si, di, r0, r1, r2, r3, zbuf, acc, isem_s, isem_d, gsem, ssem,
):
    cid = lax.axis_index("c")
    sid = lax.axis_index("s")
    nc = jnp.where(cid == 0, NC_SEG[0], NC_SEG[1])
    wbase = jnp.where(cid == 0, sid * NC_SEG[0], 16 * NC_SEG[0] + sid * NC_SEG[1])
    rows = [r0, r1, r2, r3]

    @pl.loop(0, 64)
    def _(r):
        zbuf[r, pl.ds(0, 16)] = jnp.zeros((16,), _f32)
        zbuf[r, pl.ds(16, 16)] = jnp.zeros((16,), _f32)

    @pl.loop(0, ROWS_PER_TILE // 64)
    def _(j):
        pltpu.sync_copy(zbuf, acc.at[pl.ds(sid * ROWS_PER_TILE + j * 64, 64)])

    plsc.subcore_barrier()

    def fire_idx(c, m8):
        pltpu.async_copy(src2_hbm.at[wbase + c], si.at[m8], isem_s.at[m8])
        pltpu.async_copy(dst2_hbm.at[wbase + c], di.at[m8], isem_d.at[m8])

    def wait_idx(m8):
        pltpu.make_async_copy(src2_hbm.at[0], si.at[m8], isem_s.at[m8]).wait()
        pltpu.make_async_copy(dst2_hbm.at[0], di.at[m8], isem_d.at[m8]).wait()

    def fire_gather(m8, m4):
        pltpu.async_copy(table_hbm.at[si.at[m8]], rows[m4], gsem.at[m4])

    def wait_gather(m4):
        pltpu.make_async_copy(table_hbm.at[pl.ds(0, CHUNK)], rows[m4], gsem.at[m4]).wait()

    def fire_scatter(m8, m4):
        pltpu.async_copy(rows[m4], acc.at[di.at[m8]], ssem.at[m4], add=True)

    def wait_scatter(m4):
        pltpu.make_async_copy(rows[m4], acc.at[pl.ds(0, CHUNK)], ssem.at[m4]).wait()

    fire_idx(0, 0)
    fire_idx(1, 1)
    fire_idx(2, 2)
    wait_idx(0)
    fire_gather(0, 0)
    wait_idx(1)
    fire_gather(1, 1)

    @pl.loop(0, nc, step=8)
    def _(cc):
        for b in range(8):
            c = cc + b

            @pl.when(c + 3 < nc)
            def _():
                fire_idx(c + 3, (b + 3) % 8)

            wait_gather(b % 4)
            fire_scatter(b, b % 4)

            @pl.when(c + 2 < nc)
            def _():
                wait_idx((b + 2) % 8)

                @pl.when(c >= 2)
                def _():
                    wait_scatter((b + 2) % 4)

                fire_gather((b + 2) % 8, (b + 2) % 4)

    for b in range(4):
        wait_scatter(b)

    plsc.subcore_barrier()
    pltpu.sync_copy(
        acc.at[pl.ds(sid * ROWS_PER_TILE, ROWS_PER_TILE)],
        out_hbm.at[cid, pl.ds(sid * ROWS_PER_TILE, ROWS_PER_TILE)],
    )


def _segsum(table, src2, dst2):
    k = pl.kernel(
        _segsum_body,
        out_type=jax.ShapeDtypeStruct((2, NP, 32), _f32),
        mesh=_vmesh(),
        compiler_params=_SC_PARAMS,
        scratch_types=[
            pltpu.VMEM((8, CHUNK), _i32),
            pltpu.VMEM((8, CHUNK), _i32),
            pltpu.VMEM((CHUNK, 32), _f32),
            pltpu.VMEM((CHUNK, 32), _f32),
            pltpu.VMEM((CHUNK, 32), _f32),
            pltpu.VMEM((CHUNK, 32), _f32),
            pltpu.VMEM((64, 32), _f32),
            pltpu.VMEM_SHARED((NP, 32), _f32),
            pltpu.SemaphoreType.DMA((8,)),
            pltpu.SemaphoreType.DMA((8,)),
            pltpu.SemaphoreType.DMA((4,)),
            pltpu.SemaphoreType.DMA((4,)),
        ],
    )
    return k(table, src2, dst2)


def _edge_body(
    aw_hbm, bw_hbm, src_hbm, dst_hbm, wm2_hbm, bm2_hbm, o0_hbm, o1_hbm,
    sidx_all, didx_all, ga0, ga1, gb0, gb1, ov00, ov01, ov10, ov11, wm2_v, bm2_v, wrot,
    ga_s0, ga_s1, gb_s0, gb_s1, o0_s0, o0_s1, o1_s0, o1_s1,
):
    cid = lax.axis_index("c")
    sid = lax.axis_index("s")
    nc = jnp.where(cid == 0, NC_EDGE[0], NC_EDGE[1])
    wstart = jnp.where(cid == 0, sid * NC_EDGE[0], 16 * NC_EDGE[0] + sid * NC_EDGE[1])
    ebase = wstart * CHUNK
    iot = lax.iota(_i32, 16)
    pltpu.sync_copy(wm2_hbm, wm2_v)
    pltpu.sync_copy(bm2_hbm, bm2_v)
    bv = bm2_v[...]

    @pl.loop(0, 32)
    def _(k):
        kl = jnp.bitwise_and(k + iot, 31)
        z16 = jnp.zeros((16,), _i32)
        o16 = jnp.ones((16,), _i32)
        wrot[0, k] = plsc.load_gather(wm2_v, [z16, kl]) * (1.0 / QS)
        wrot[1, k] = plsc.load_gather(wm2_v, [z16, kl + 32]) * (1.0 / QS)
        wrot[2, k] = plsc.load_gather(wm2_v, [o16, kl]) * (1.0 / QS)
        wrot[3, k] = plsc.load_gather(wm2_v, [o16, kl + 32]) * (1.0 / QS)

    pltpu.sync_copy(src_hbm.at[pl.ds(ebase, IDX_PAD)], sidx_all)
    pltpu.sync_copy(dst_hbm.at[pl.ds(ebase, IDX_PAD)], didx_all)
    ga = [ga0, ga1]
    gb = [gb0, gb1]
    ov0 = [ov00, ov01]
    ov1 = [ov10, ov11]
    gasem = [ga_s0, ga_s1]
    gbsem = [gb_s0, gb_s1]
    o0sem = [o0_s0, o0_s1]
    o1sem = [o1_s0, o1_s1]
    rows_g = [g * 16 + iot for g in range(8)]

    def fire_gather(c, p):
        pltpu.async_copy(
            aw_hbm.at[sidx_all.at[pl.ds(c * CHUNK, CHUNK)]], ga[p], gasem[p]
        )
        pltpu.async_copy(
            bw_hbm.at[didx_all.at[pl.ds(c * CHUNK, CHUNK)]], gb[p], gbsem[p]
        )

    def wait_gather(p):
        pltpu.make_async_copy(aw_hbm.at[pl.ds(0, CHUNK)], ga[p], gasem[p]).wait()
        pltpu.make_async_copy(bw_hbm.at[pl.ds(0, CHUNK)], gb[p], gbsem[p]).wait()

    def wait_out(p):
        pltpu.make_async_copy(ov0[p], o0_hbm.at[pl.ds(0, CHUNK)], o0sem[p]).wait()
        pltpu.make_async_copy(ov1[p], o1_hbm.at[pl.ds(0, CHUNK)], o1sem[p]).wait()

    fire_gather(0, 0)
    fire_gather(1, 1)

    @pl.loop(0, nc, step=2)
    def _(cc):
        for b in range(2):
            c = cc + b
            p = b
            wait_gather(p)

            @pl.when(c >= 2)
            def _():
                wait_out(p)

            init = tuple(jnp.zeros((16,), _f32) + bv[0] for _ in range(8)) + tuple(
                jnp.zeros((16,), _f32) + bv[1] for _ in range(8)
            )

            @pl.loop(0, 32, init_carry=init)
            def accs(k, carry):
                kl = jnp.bitwise_and(k + iot, 31)
                w0h = wrot[0, k]
                w0l = wrot[1, k]
                w1h = wrot[2, k]
                w1l = wrot[3, k]
                out = []
                out1 = []
                for g in range(8):
                    wa = plsc.load_gather(ga[p], [rows_g[g], kl])
                    wb = plsc.load_gather(gb[p], [rows_g[g], kl])
                    sh = lax.shift_right_arithmetic(wa, 16) + lax.shift_right_arithmetic(wb, 16)
                    sl = lax.shift_right_arithmetic(
                        lax.shift_left(wa, 16), 16
                    ) + lax.shift_right_arithmetic(lax.shift_left(wb, 16), 16)
                    rh = jnp.maximum(sh, 0).astype(_f32)
                    rl = jnp.maximum(sl, 0).astype(_f32)
                    out.append(carry[g] + rh * w0h + rl * w0l)
                    out1.append(carry[8 + g] + rh * w1h + rl * w1l)
                return tuple(out) + tuple(out1)

            for g in range(8):
                ov0[p][pl.ds(g * 16, 16)] = accs[g]
                ov1[p][pl.ds(g * 16, 16)] = accs[8 + g]

            base = ebase + c * CHUNK
            pltpu.async_copy(ov0[p], o0_hbm.at[pl.ds(base, CHUNK)], o0sem[p])
            pltpu.async_copy(ov1[p], o1_hbm.at[pl.ds(base, CHUNK)], o1sem[p])

            @pl.when(c + 2 < nc)
            def _():
                fire_gather(c + 2, p)

    for b in range(2):
        wait_out(b)


def _edge_mlp(aw, bw, srcp, dstp, wm2, bm2):
    k = pl.kernel(
        _edge_body,
        out_type=(
            jax.ShapeDtypeStruct((EP,), _f32),
            jax.ShapeDtypeStruct((EP,), _f32),
        ),
        mesh=_vmesh(),
        compiler_params=_SC_PARAMS,
        scratch_types=[
            pltpu.VMEM((IDX_PAD,), _i32),
            pltpu.VMEM((IDX_PAD,), _i32),
            pltpu.VMEM((CHUNK, 32), _i32),
            pltpu.VMEM((CHUNK, 32), _i32),
            pltpu.VMEM((CHUNK, 32), _i32),
            pltpu.VMEM((CHUNK, 32), _i32),
            pltpu.VMEM((CHUNK,), _f32),
            pltpu.VMEM((CHUNK,), _f32),
            pltpu.VMEM((CHUNK,), _f32),
            pltpu.VMEM((CHUNK,), _f32),
            pltpu.VMEM((2, 64), _f32),
            pltpu.VMEM((16,), _f32),
            pltpu.VMEM((4, 32, 16), _f32),
        ] + [pltpu.SemaphoreType.DMA] * 8,
    )
    return k(aw, bw, srcp, dstp, wm2, bm2)


_BLK = 1024
_GRID = NP // _BLK


def _dot(a, b):
    return lax.dot_general(a, b, (((1,), (0,)), ((), ())))


def _tc1_body(acc_ref, h0_ref, w1l_ref, w1r_ref, b1_ref, h1a_ref, h1b_ref, rcp_ref):
    acc = acc_ref[0] + acc_ref[1]
    cnt = jnp.maximum(acc[:, 25:26], 1.0)
    mean = acc / cnt
    out = _dot(mean, w1l_ref[...]) + b1_ref[...] + _dot(h0_ref[...], w1r_ref[...])
    norm = jnp.sqrt(jnp.sum(out * out, axis=1, keepdims=True))
    h = jnp.maximum(out / jnp.maximum(norm, 1e-12), 0.0)
    h1a_ref[...] = h[:, :32]
    h1b_ref[...] = h[:, 32:]
    rcp_ref[...] = jnp.broadcast_to(cnt, (_BLK, 8))


def _tc1(acc, h0, w1lT, w1rT, b1):
    return pl.pallas_call(
        _tc1_body,
        grid=(_GRID,),
        in_specs=[
            pl.BlockSpec((2, _BLK, 32), lambda i: (0, i, 0)),
            pl.BlockSpec((_BLK, 32), lambda i: (i, 0)),
            pl.BlockSpec((32, 64), lambda i: (0, 0)),
            pl.BlockSpec((32, 64), lambda i: (0, 0)),
            pl.BlockSpec((1, 64), lambda i: (0, 0)),
        ],
        out_specs=[
            pl.BlockSpec((_BLK, 32), lambda i: (i, 0)),
            pl.BlockSpec((_BLK, 32), lambda i: (i, 0)),
            pl.BlockSpec((_BLK, 8), lambda i: (i, 0)),
        ],
        out_shape=[
            jax.ShapeDtypeStruct((NP, 32), _f32),
            jax.ShapeDtypeStruct((NP, 32), _f32),
            jax.ShapeDtypeStruct((NP, 8), _f32),
        ],
    )(acc, h0, w1lT, w1rT, b1)


def _tc2_body(
    acca_ref, accb_ref, h1a_ref, h1b_ref, rcp_ref,
    w2l_ref, w2r_ref, b2_ref, wm1s_ref, wm1d_ref, bm1_ref,
    aw_ref, bw_ref,
):
    cnt = rcp_ref[:, 0:1]
    mean = jnp.concatenate(
        [(acca_ref[0] + acca_ref[1]) / cnt, (accb_ref[0] + accb_ref[1]) / cnt], axis=1
    )
    h1 = jnp.concatenate([h1a_ref[...], h1b_ref[...]], axis=1)
    out = _dot(mean, w2l_ref[...]) + b2_ref[...] + _dot(h1, w2r_ref[...])
    norm = jnp.sqrt(jnp.sum(out * out, axis=1, keepdims=True))
    h2 = jnp.maximum(out / jnp.maximum(norm, 1e-12), 0.0)
    af = _dot(h2, wm1s_ref[...])
    bf = _dot(h2, wm1d_ref[...]) + bm1_ref[...]

    def pack(x):
        q = lax.convert_element_type(
            lax.round(jnp.clip(x, -1.6, 1.6) * QS), jnp.int32
        )
        return (q[:, :32] << 16) | (q[:, 32:] & 0xFFFF)

    aw_ref[...] = pack(af)
    bw_ref[...] = pack(bf)


def _tc2(acca, accb, h1a, h1b, rcp8, w2lT, w2rT, b2, wm1sT, wm1dT, bm1):
    return pl.pallas_call(
        _tc2_body,
        grid=(_GRID,),
        in_specs=[
            pl.BlockSpec((2, _BLK, 32), lambda i: (0, i, 0)),
            pl.BlockSpec((2, _BLK, 32), lambda i: (0, i, 0)),
            pl.BlockSpec((_BLK, 32), lambda i: (i, 0)),
            pl.BlockSpec((_BLK, 32), lambda i: (i, 0)),
            pl.BlockSpec((_BLK, 8), lambda i: (i, 0)),
            pl.BlockSpec((64, 64), lambda i: (0, 0)),
            pl.BlockSpec((64, 64), lambda i: (0, 0)),
            pl.BlockSpec((1, 64), lambda i: (0, 0)),
            pl.BlockSpec((64, 64), lambda i: (0, 0)),
            pl.BlockSpec((64, 64), lambda i: (0, 0)),
            pl.BlockSpec((1, 64), lambda i: (0, 0)),
        ],
        out_specs=[
            pl.BlockSpec((_BLK, 32), lambda i: (i, 0)),
            pl.BlockSpec((_BLK, 32), lambda i: (i, 0)),
        ],
        out_shape=[
            jax.ShapeDtypeStruct((NP, 32), jnp.int32),
            jax.ShapeDtypeStruct((NP, 32), jnp.int32),
        ],
    )(acca, accb, h1a, h1b, rcp8, w2lT, w2rT, b2, wm1sT, wm1dT, bm1)


def kernel(x, edge_index, pred_embed, W1l, b1l, W1r, W2l, b2l, W2r, Wm1, bm1, Wm2, bm2):
    xp = jnp.zeros((NP, 16), _f32).at[:N, :10].set(x)
    pad = jnp.full((EPI - E,), N, _i32)
    srcp = jnp.concatenate([edge_index[0], pad])
    dstp = jnp.concatenate([edge_index[1], pad])

    w1lT = jnp.zeros((32, 64), _f32).at[:25].set(W1l.T)
    w1rT = jnp.zeros((32, 64), _f32).at[:25].set(W1r.T)

    src2 = srcp.reshape(EPI // CHUNK, CHUNK)
    dst2 = dstp.reshape(EPI // CHUNK, CHUNK)

    h0 = _build_h0(xp, pred_embed)
    acc1 = _segsum(h0, src2, dst2)
    h1a, h1b, rcp8 = _tc1(acc1, h0, w1lT, w1rT, b1l.reshape(1, 64))
    acca = _segsum(h1a, src2, dst2)
    accb = _segsum(h1b, src2, dst2)
    aw, bw = _tc2(
        acca, accb, h1a, h1b, rcp8,
        W2l.T, W2r.T, b2l.reshape(1, 64),
        Wm1[:, :64].T, Wm1[:, 64:].T, bm1.reshape(1, 64),
    )
    o0, o1 = _edge_mlp(
        aw, bw, srcp, dstp, Wm2, jnp.zeros((16,), _f32).at[:2].set(bm2)
    )
    return jnp.stack([o0[:E], o1[:E]], axis=1)


# speedup vs baseline: 6.0140x; 1.0235x over previous
import functools

import jax
import jax.numpy as jnp
from jax import lax
from jax.experimental import pallas as pl
from jax.experimental.pallas import tpu as pltpu
from jax.experimental.pallas import tpu_sc as plsc

N = 50000
E = 800000
NP = 53248
EP = 819200
CHUNK = 128
NWORK = 32
EDGES_PER_W = EP // NWORK
NCHUNK_E = EDGES_PER_W // CHUNK
NC_EDGE = (264, 136)
NC_SEG = (304, 96)
NCMAX_EDGE = max(NC_EDGE)
QS = 20000.0
IDX_PAD = NCMAX_EDGE * CHUNK
EPI = (16 * NC_EDGE[0] + 15 * NC_EDGE[1]) * CHUNK + IDX_PAD
NODE_CHUNKS = NP // CHUNK
ROWS_PER_TILE = NP // 16

_f32 = jnp.float32
_i32 = jnp.int32


def _vmesh():
    return plsc.VectorSubcoreMesh(
        core_axis_name="c", subcore_axis_name="s", num_cores=2, num_subcores=16
    )


_SC_PARAMS = pltpu.CompilerParams(
    needs_layout_passes=False, use_tc_tiling_on_sc=False
)


def _worker_id():
    return lax.axis_index("c") * 16 + lax.axis_index("s")


def _build_h0_body(
    xp_hbm, pred_hbm, h0_hbm, xv0, xv1, pv, idxv, hv0, hv1, xs0, xs1, hs0, hs1
):
    w = _worker_id()
    iot = lax.iota(_i32, 16)
    xv = [xv0, xv1]
    hv = [hv0, hv1]
    xsem = [xs0, xs1]
    hsem = [hs0, hs1]
    cpw = NODE_CHUNKS // NWORK
    c0 = w * cpw

    def fire_x(i, p):
        pltpu.async_copy(xp_hbm.at[pl.ds((c0 + i) * CHUNK, CHUNK)], xv[p], xsem[p])

    def wait_x(p):
        pltpu.make_async_copy(xp_hbm.at[pl.ds(0, CHUNK)], xv[p], xsem[p]).wait()

    def wait_h(p):
        pltpu.make_async_copy(hv[p], h0_hbm.at[pl.ds(0, CHUNK)], hsem[p]).wait()

    def stage(i, p, first, last):
        wait_x(p)
        if not last:
            fire_x(i + 1, 1 - p)
        for g in range(8):
            rows = g * 16 + iot
            pidf = plsc.load_gather(xv[p], [rows, jnp.full((16,), 1, _i32)])
            plsc.store_scatter(idxv, [rows], pidf.astype(_i32))
        pltpu.sync_copy(pred_hbm.at[idxv], pv)
        if not first:
            wait_h(p)
        for g in range(8):
            rows = g * 16 + iot
            v = plsc.load_gather(xv[p], [rows, jnp.full((16,), 0, _i32)])
            plsc.store_scatter(hv[p], [rows, jnp.full((16,), 0, _i32)], v)
            for j in range(8):
                v = plsc.load_gather(xv[p], [rows, jnp.full((16,), 2 + j, _i32)])
                plsc.store_scatter(hv[p], [rows, jnp.full((16,), 1 + j, _i32)], v)
            for j in range(16):
                v = plsc.load_gather(pv, [rows, jnp.full((16,), j, _i32)])
                plsc.store_scatter(hv[p], [rows, jnp.full((16,), 9 + j, _i32)], v)
            plsc.store_scatter(
                hv[p], [rows, jnp.full((16,), 25, _i32)], jnp.ones((16,), _f32)
            )
            for j in range(26, 32):
                plsc.store_scatter(
                    hv[p], [rows, jnp.full((16,), j, _i32)], jnp.zeros((16,), _f32)
                )
        pltpu.async_copy(hv[p], h0_hbm.at[pl.ds((c0 + i) * CHUNK, CHUNK)], hsem[p])

    fire_x(0, 0)
    stage(0, 0, first=True, last=False)
    stage(1, 1, first=True, last=False)

    @pl.loop(2, cpw - 1, step=2)
    def _(ii):
        for b in range(2):
            stage(ii + b, b, first=False, last=False)

    stage(cpw - 1, 0, first=False, last=True)
    wait_h(0)
    wait_h(1)


def _build_h0(xp, pred):
    k = pl.kernel(
        _build_h0_body,
        out_type=jax.ShapeDtypeStruct((NP, 32), _f32),
        mesh=_vmesh(),
        compiler_params=_SC_PARAMS,
        scratch_types=[
            pltpu.VMEM((CHUNK, 16), _f32),
            pltpu.VMEM((CHUNK, 16), _f32),
            pltpu.VMEM((CHUNK, 16), _f32),
            pltpu.VMEM((CHUNK,), _i32),
            pltpu.VMEM((CHUNK, 32), _f32),
            pltpu.VMEM((CHUNK, 32), _f32),
        ] + [pltpu.SemaphoreType.DMA] * 4,
    )
    return k(xp, pred)


def _segsum_body(
    table_hbm, src2_hbm, dst2_hbm, out_hbm,
    si, di, r0, r1, r2, r3, zbuf, acc, isem_s, isem_d, gsem, ssem,
):
    cid = lax.axis_index("c")
    sid = lax.axis_index("s")
    nc = jnp.where(cid == 0, NC_SEG[0], NC_SEG[1])
    wbase = jnp.where(cid == 0, sid * NC_SEG[0], 16 * NC_SEG[0] + sid * NC_SEG[1])
    rows = [r0, r1, r2, r3]

    @pl.loop(0, 64)
    def _(r):
        zbuf[r, pl.ds(0, 16)] = jnp.zeros((16,), _f32)
        zbuf[r, pl.ds(16, 16)] = jnp.zeros((16,), _f32)

    @pl.loop(0, ROWS_PER_TILE // 64)
    def _(j):
        pltpu.async_copy(zbuf, acc.at[pl.ds(sid * ROWS_PER_TILE + j * 64, 64)], ssem.at[0])

    @pl.loop(0, ROWS_PER_TILE // 64)
    def _(j):
        pltpu.make_async_copy(zbuf, acc.at[pl.ds(0, 64)], ssem.at[0]).wait()

    plsc.subcore_barrier()

    def fire_idx(c, m8):
        pltpu.async_copy(src2_hbm.at[wbase + c], si.at[m8], isem_s.at[m8])
        pltpu.async_copy(dst2_hbm.at[wbase + c], di.at[m8], isem_d.at[m8])

    def wait_idx(m8):
        pltpu.make_async_copy(src2_hbm.at[0], si.at[m8], isem_s.at[m8]).wait()
        pltpu.make_async_copy(dst2_hbm.at[0], di.at[m8], isem_d.at[m8]).wait()

    def fire_gather(m8, m4):
        pltpu.async_copy(table_hbm.at[si.at[m8]], rows[m4], gsem.at[m4])

    def wait_gather(m4):
        pltpu.make_async_copy(table_hbm.at[pl.ds(0, CHUNK)], rows[m4], gsem.at[m4]).wait()

    def fire_scatter(m8, m4):
        pltpu.async_copy(rows[m4], acc.at[di.at[m8]], ssem.at[m4], add=True)

    def wait_scatter(m4):
        pltpu.make_async_copy(rows[m4], acc.at[pl.ds(0, CHUNK)], ssem.at[m4]).wait()

    fire_idx(0, 0)
    fire_idx(1, 1)
    fire_idx(2, 2)
    wait_idx(0)
    fire_gather(0, 0)
    wait_idx(1)
    fire_gather(1, 1)

    @pl.loop(0, nc, step=8)
    def _(cc):
        for b in range(8):
            c = cc + b

            @pl.when(c + 3 < nc)
            def _():
                fire_idx(c + 3, (b + 3) % 8)

            wait_gather(b % 4)
            fire_scatter(b, b % 4)

            @pl.when(c + 2 < nc)
            def _():
                wait_idx((b + 2) % 8)

                @pl.when(c >= 2)
                def _():
                    wait_scatter((b + 2) % 4)

                fire_gather((b + 2) % 8, (b + 2) % 4)

    for b in range(4):
        wait_scatter(b)

    plsc.subcore_barrier()
    pltpu.sync_copy(
        acc.at[pl.ds(sid * ROWS_PER_TILE, ROWS_PER_TILE)],
        out_hbm.at[cid, pl.ds(sid * ROWS_PER_TILE, ROWS_PER_TILE)],
    )


def _segsum(table, src2, dst2):
    k = pl.kernel(
        _segsum_body,
        out_type=jax.ShapeDtypeStruct((2, NP, 32), _f32),
        mesh=_vmesh(),
        compiler_params=_SC_PARAMS,
        scratch_types=[
            pltpu.VMEM((8, CHUNK), _i32),
            pltpu.VMEM((8, CHUNK), _i32),
            pltpu.VMEM((CHUNK, 32), _f32),
            pltpu.VMEM((CHUNK, 32), _f32),
            pltpu.VMEM((CHUNK, 32), _f32),
            pltpu.VMEM((CHUNK, 32), _f32),
            pltpu.VMEM((64, 32), _f32),
            pltpu.VMEM_SHARED((NP, 32), _f32),
            pltpu.SemaphoreType.DMA((8,)),
            pltpu.SemaphoreType.DMA((8,)),
            pltpu.SemaphoreType.DMA((4,)),
            pltpu.SemaphoreType.DMA((4,)),
        ],
    )
    return k(table, src2, dst2)


def _edge_body(
    aw_hbm, bw_hbm, src_hbm, dst_hbm, wm2_hbm, bm2_hbm, o0_hbm, o1_hbm,
    sidx_all, didx_all, ga0, ga1, gb0, gb1, ov00, ov01, ov10, ov11, wm2_v, bm2_v, wrot,
    ga_s0, ga_s1, gb_s0, gb_s1, o0_s0, o0_s1, o1_s0, o1_s1,
):
    cid = lax.axis_index("c")
    sid = lax.axis_index("s")
    nc = jnp.where(cid == 0, NC_EDGE[0], NC_EDGE[1])
    wstart = jnp.where(cid == 0, sid * NC_EDGE[0], 16 * NC_EDGE[0] + sid * NC_EDGE[1])
    ebase = wstart * CHUNK
    iot = lax.iota(_i32, 16)
    pltpu.sync_copy(wm2_hbm, wm2_v)
    pltpu.sync_copy(bm2_hbm, bm2_v)
    bv = bm2_v[...]

    @pl.loop(0, 32)
    def _(k):
        kl = jnp.bitwise_and(k + iot, 31)
        z16 = jnp.zeros((16,), _i32)
        o16 = jnp.ones((16,), _i32)
        wrot[0, k] = plsc.load_gather(wm2_v, [z16, kl]) * (1.0 / QS)
        wrot[1, k] = plsc.load_gather(wm2_v, [z16, kl + 32]) * (1.0 / QS)
        wrot[2, k] = plsc.load_gather(wm2_v, [o16, kl]) * (1.0 / QS)
        wrot[3, k] = plsc.load_gather(wm2_v, [o16, kl + 32]) * (1.0 / QS)

    pltpu.sync_copy(src_hbm.at[pl.ds(ebase, IDX_PAD)], sidx_all)
    pltpu.sync_copy(dst_hbm.at[pl.ds(ebase, IDX_PAD)], didx_all)
    ga = [ga0, ga1]
    gb = [gb0, gb1]
    ov0 = [ov00, ov01]
    ov1 = [ov10, ov11]
    gasem = [ga_s0, ga_s1]
    gbsem = [gb_s0, gb_s1]
    o0sem = [o0_s0, o0_s1]
    o1sem = [o1_s0, o1_s1]
    rows_g = [g * 16 + iot for g in range(8)]

    def fire_gather(c, p):
        pltpu.async_copy(
            aw_hbm.at[sidx_all.at[pl.ds(c * CHUNK, CHUNK)]], ga[p], gasem[p]
        )
        pltpu.async_copy(
            bw_hbm.at[didx_all.at[pl.ds(c * CHUNK, CHUNK)]], gb[p], gbsem[p]
        )

    def wait_gather(p):
        pltpu.make_async_copy(aw_hbm.at[pl.ds(0, CHUNK)], ga[p], gasem[p]).wait()
        pltpu.make_async_copy(bw_hbm.at[pl.ds(0, CHUNK)], gb[p], gbsem[p]).wait()

    def wait_out(p):
        pltpu.make_async_copy(ov0[p], o0_hbm.at[pl.ds(0, CHUNK)], o0sem[p]).wait()
        pltpu.make_async_copy(ov1[p], o1_hbm.at[pl.ds(0, CHUNK)], o1sem[p]).wait()

    fire_gather(0, 0)
    fire_gather(1, 1)

    @pl.loop(0, nc, step=2)
    def _(cc):
        for b in range(2):
            c = cc + b
            p = b
            wait_gather(p)

            @pl.when(c >= 2)
            def _():
                wait_out(p)

            init = tuple(jnp.zeros((16,), _f32) + bv[0] for _ in range(8)) + tuple(
                jnp.zeros((16,), _f32) + bv[1] for _ in range(8)
            )

            @pl.loop(0, 32, init_carry=init)
            def accs(k, carry):
                kl = jnp.bitwise_and(k + iot, 31)
                w0h = wrot[0, k]
                w0l = wrot[1, k]
                w1h = wrot[2, k]
                w1l = wrot[3, k]
                out = []
                out1 = []
                for g in range(8):
                    wa = plsc.load_gather(ga[p], [rows_g[g], kl])
                    wb = plsc.load_gather(gb[p], [rows_g[g], kl])
                    sh = lax.shift_right_arithmetic(wa, 16) + lax.shift_right_arithmetic(wb, 16)
                    sl = lax.shift_right_arithmetic(
                        lax.shift_left(wa, 16), 16
                    ) + lax.shift_right_arithmetic(lax.shift_left(wb, 16), 16)
                    rh = jnp.maximum(sh, 0).astype(_f32)
                    rl = jnp.maximum(sl, 0).astype(_f32)
                    out.append(carry[g] + rh * w0h + rl * w0l)
                    out1.append(carry[8 + g] + rh * w1h + rl * w1l)
                return tuple(out) + tuple(out1)

            for g in range(8):
                ov0[p][pl.ds(g * 16, 16)] = accs[g]
                ov1[p][pl.ds(g * 16, 16)] = accs[8 + g]

            base = ebase + c * CHUNK
            pltpu.async_copy(ov0[p], o0_hbm.at[pl.ds(base, CHUNK)], o0sem[p])
            pltpu.async_copy(ov1[p], o1_hbm.at[pl.ds(base, CHUNK)], o1sem[p])

            @pl.when(c + 2 < nc)
            def _():
                fire_gather(c + 2, p)

    for b in range(2):
        wait_out(b)


def _edge_mlp(aw, bw, srcp, dstp, wm2, bm2):
    k = pl.kernel(
        _edge_body,
        out_type=(
            jax.ShapeDtypeStruct((EP,), _f32),
            jax.ShapeDtypeStruct((EP,), _f32),
        ),
        mesh=_vmesh(),
        compiler_params=_SC_PARAMS,
        scratch_types=[
            pltpu.VMEM((IDX_PAD,), _i32),
            pltpu.VMEM((IDX_PAD,), _i32),
            pltpu.VMEM((CHUNK, 32), _i32),
            pltpu.VMEM((CHUNK, 32), _i32),
            pltpu.VMEM((CHUNK, 32), _i32),
            pltpu.VMEM((CHUNK, 32), _i32),
            pltpu.VMEM((CHUNK,), _f32),
            pltpu.VMEM((CHUNK,), _f32),
            pltpu.VMEM((CHUNK,), _f32),
            pltpu.VMEM((CHUNK,), _f32),
            pltpu.VMEM((2, 64), _f32),
            pltpu.VMEM((16,), _f32),
            pltpu.VMEM((4, 32, 16), _f32),
        ] + [pltpu.SemaphoreType.DMA] * 8,
    )
    return k(aw, bw, srcp, dstp, wm2, bm2)


_BLK = 1024
_GRID = NP // _BLK


def _dot(a, b):
    return lax.dot_general(a, b, (((1,), (0,)), ((), ())))


def _tc1_body(acc_ref, h0_ref, w1l_ref, w1r_ref, b1_ref, h1a_ref, h1b_ref, rcp_ref):
    acc = acc_ref[0] + acc_ref[1]
    cnt = jnp.maximum(acc[:, 25:26], 1.0)
    mean = acc / cnt
    out = _dot(mean, w1l_ref[...]) + b1_ref[...] + _dot(h0_ref[...], w1r_ref[...])
    norm = jnp.sqrt(jnp.sum(out * out, axis=1, keepdims=True))
    h = jnp.maximum(out / jnp.maximum(norm, 1e-12), 0.0)
    h1a_ref[...] = h[:, :32]
    h1b_ref[...] = h[:, 32:]
    rcp_ref[...] = jnp.broadcast_to(cnt, (_BLK, 8))


def _tc1(acc, h0, w1lT, w1rT, b1):
    return pl.pallas_call(
        _tc1_body,
        grid=(_GRID,),
        in_specs=[
            pl.BlockSpec((2, _BLK, 32), lambda i: (0, i, 0)),
            pl.BlockSpec((_BLK, 32), lambda i: (i, 0)),
            pl.BlockSpec((32, 64), lambda i: (0, 0)),
            pl.BlockSpec((32, 64), lambda i: (0, 0)),
            pl.BlockSpec((1, 64), lambda i: (0, 0)),
        ],
        out_specs=[
            pl.BlockSpec((_BLK, 32), lambda i: (i, 0)),
            pl.BlockSpec((_BLK, 32), lambda i: (i, 0)),
            pl.BlockSpec((_BLK, 8), lambda i: (i, 0)),
        ],
        out_shape=[
            jax.ShapeDtypeStruct((NP, 32), _f32),
            jax.ShapeDtypeStruct((NP, 32), _f32),
            jax.ShapeDtypeStruct((NP, 8), _f32),
        ],
    )(acc, h0, w1lT, w1rT, b1)


def _tc2_body(
    acca_ref, accb_ref, h1a_ref, h1b_ref, rcp_ref,
    w2l_ref, w2r_ref, b2_ref, wm1s_ref, wm1d_ref, bm1_ref,
    aw_ref, bw_ref,
):
    cnt = rcp_ref[:, 0:1]
    mean = jnp.concatenate(
        [(acca_ref[0] + acca_ref[1]) / cnt, (accb_ref[0] + accb_ref[1]) / cnt], axis=1
    )
    h1 = jnp.concatenate([h1a_ref[...], h1b_ref[...]], axis=1)
    out = _dot(mean, w2l_ref[...]) + b2_ref[...] + _dot(h1, w2r_ref[...])
    norm = jnp.sqrt(jnp.sum(out * out, axis=1, keepdims=True))
    h2 = jnp.maximum(out / jnp.maximum(norm, 1e-12), 0.0)
    af = _dot(h2, wm1s_ref[...])
    bf = _dot(h2, wm1d_ref[...]) + bm1_ref[...]

    def pack(x):
        q = lax.convert_element_type(
            lax.round(jnp.clip(x, -1.6, 1.6) * QS), jnp.int32
        )
        return (q[:, :32] << 16) | (q[:, 32:] & 0xFFFF)

    aw_ref[...] = pack(af)
    bw_ref[...] = pack(bf)


def _tc2(acca, accb, h1a, h1b, rcp8, w2lT, w2rT, b2, wm1sT, wm1dT, bm1):
    return pl.pallas_call(
        _tc2_body,
        grid=(_GRID,),
        in_specs=[
            pl.BlockSpec((2, _BLK, 32), lambda i: (0, i, 0)),
            pl.BlockSpec((2, _BLK, 32), lambda i: (0, i, 0)),
            pl.BlockSpec((_BLK, 32), lambda i: (i, 0)),
            pl.BlockSpec((_BLK, 32), lambda i: (i, 0)),
            pl.BlockSpec((_BLK, 8), lambda i: (i, 0)),
            pl.BlockSpec((64, 64), lambda i: (0, 0)),
            pl.BlockSpec((64, 64), lambda i: (0, 0)),
            pl.BlockSpec((1, 64), lambda i: (0, 0)),
            pl.BlockSpec((64, 64), lambda i: (0, 0)),
            pl.BlockSpec((64, 64), lambda i: (0, 0)),
            pl.BlockSpec((1, 64), lambda i: (0, 0)),
        ],
        out_specs=[
            pl.BlockSpec((_BLK, 32), lambda i: (i, 0)),
            pl.BlockSpec((_BLK, 32), lambda i: (i, 0)),
        ],
        out_shape=[
            jax.ShapeDtypeStruct((NP, 32), jnp.int32),
            jax.ShapeDtypeStruct((NP, 32), jnp.int32),
        ],
    )(acca, accb, h1a, h1b, rcp8, w2lT, w2rT, b2, wm1sT, wm1dT, bm1)


def kernel(x, edge_index, pred_embed, W1l, b1l, W1r, W2l, b2l, W2r, Wm1, bm1, Wm2, bm2):
    xp = jnp.zeros((NP, 16), _f32).at[:N, :10].set(x)
    pad = jnp.full((EPI - E,), N, _i32)
    srcp = jnp.concatenate([edge_index[0], pad])
    dstp = jnp.concatenate([edge_index[1], pad])

    w1lT = jnp.zeros((32, 64), _f32).at[:25].set(W1l.T)
    w1rT = jnp.zeros((32, 64), _f32).at[:25].set(W1r.T)

    src2 = srcp.reshape(EPI // CHUNK, CHUNK)
    dst2 = dstp.reshape(EPI // CHUNK, CHUNK)

    h0 = _build_h0(xp, pred_embed)
    acc1 = _segsum(h0, src2, dst2)
    h1a, h1b, rcp8 = _tc1(acc1, h0, w1lT, w1rT, b1l.reshape(1, 64))
    acca = _segsum(h1a, src2, dst2)
    accb = _segsum(h1b, src2, dst2)
    aw, bw = _tc2(
        acca, accb, h1a, h1b, rcp8,
        W2l.T, W2r.T, b2l.reshape(1, 64),
        Wm1[:, :64].T, Wm1[:, 64:].T, bm1.reshape(1, 64),
    )
    o0, o1 = _edge_mlp(
        aw, bw, srcp, dstp, Wm2, jnp.zeros((16,), _f32).at[:2].set(bm2)
    )
    return jnp.stack([o0[:E], o1[:E]], axis=1)


# speedup vs baseline: 6.8513x; 1.1392x over previous
import functools

import jax
import jax.numpy as jnp
from jax import lax
from jax.experimental import pallas as pl
from jax.experimental.pallas import tpu as pltpu
from jax.experimental.pallas import tpu_sc as plsc

N = 50000
E = 800000
NP = 53248
EP = 819200
CHUNK = 128
NWORK = 32
EDGES_PER_W = EP // NWORK
NCHUNK_E = EDGES_PER_W // CHUNK
NC_EDGE = (240, 160)
NC_SEG = (304, 96)
NCMAX_EDGE = max(NC_EDGE)
QS = 20000.0
IDX_PAD = NCMAX_EDGE * CHUNK
EPI = (16 * NC_EDGE[0] + 15 * NC_EDGE[1]) * CHUNK + IDX_PAD
NODE_CHUNKS = NP // CHUNK
ROWS_PER_TILE = NP // 16

_f32 = jnp.float32
_i32 = jnp.int32


def _vmesh():
    return plsc.VectorSubcoreMesh(
        core_axis_name="c", subcore_axis_name="s", num_cores=2, num_subcores=16
    )


_SC_PARAMS = pltpu.CompilerParams(
    needs_layout_passes=False, use_tc_tiling_on_sc=False
)


def _worker_id():
    return lax.axis_index("c") * 16 + lax.axis_index("s")


def _build_h0_body(
    xp_hbm, pred_hbm, h0_hbm, xv0, xv1, pred_v, hv0, hv1, xs0, xs1, hs0, hs1
):
    w = _worker_id()
    iot = lax.iota(_i32, 16)
    xv = [xv0, xv1]
    hv = [hv0, hv1]
    xsem = [xs0, xs1]
    hsem = [hs0, hs1]
    cpw = NODE_CHUNKS // NWORK
    c0 = w * cpw
    pltpu.sync_copy(pred_hbm, pred_v)

    def fire_x(i, p):
        pltpu.async_copy(xp_hbm.at[pl.ds((c0 + i) * CHUNK, CHUNK)], xv[p], xsem[p])

    def wait_x(p):
        pltpu.make_async_copy(xp_hbm.at[pl.ds(0, CHUNK)], xv[p], xsem[p]).wait()

    def wait_h(p):
        pltpu.make_async_copy(hv[p], h0_hbm.at[pl.ds(0, CHUNK)], hsem[p]).wait()

    def stage(i, p, first, last):
        wait_x(p)
        if not last:
            fire_x(i + 1, 1 - p)
        if not first:
            wait_h(p)
        for g in range(8):
            rows = g * 16 + iot
            pid = plsc.load_gather(xv[p], [rows, jnp.full((16,), 1, _i32)]).astype(_i32)
            v = plsc.load_gather(xv[p], [rows, jnp.full((16,), 0, _i32)])
            plsc.store_scatter(hv[p], [rows, jnp.full((16,), 0, _i32)], v)
            for j in range(8):
                v = plsc.load_gather(xv[p], [rows, jnp.full((16,), 2 + j, _i32)])
                plsc.store_scatter(hv[p], [rows, jnp.full((16,), 1 + j, _i32)], v)
            for j in range(16):
                v = plsc.load_gather(pred_v, [pid, jnp.full((16,), j, _i32)])
                plsc.store_scatter(hv[p], [rows, jnp.full((16,), 9 + j, _i32)], v)
            plsc.store_scatter(
                hv[p], [rows, jnp.full((16,), 25, _i32)], jnp.ones((16,), _f32)
            )
            for j in range(26, 32):
                plsc.store_scatter(
                    hv[p], [rows, jnp.full((16,), j, _i32)], jnp.zeros((16,), _f32)
                )
        pltpu.async_copy(hv[p], h0_hbm.at[pl.ds((c0 + i) * CHUNK, CHUNK)], hsem[p])

    fire_x(0, 0)
    stage(0, 0, first=True, last=False)
    stage(1, 1, first=True, last=False)

    @pl.loop(2, cpw - 1, step=2)
    def _(ii):
        for b in range(2):
            stage(ii + b, b, first=False, last=False)

    stage(cpw - 1, 0, first=False, last=True)
    wait_h(0)
    wait_h(1)


def _build_h0(xp, pred):
    k = pl.kernel(
        _build_h0_body,
        out_type=jax.ShapeDtypeStruct((NP, 32), _f32),
        mesh=_vmesh(),
        compiler_params=_SC_PARAMS,
        scratch_types=[
            pltpu.VMEM((CHUNK, 16), _f32),
            pltpu.VMEM((CHUNK, 16), _f32),
            pltpu.VMEM((1001, 16), _f32),
            pltpu.VMEM((CHUNK, 32), _f32),
            pltpu.VMEM((CHUNK, 32), _f32),
        ] + [pltpu.SemaphoreType.DMA] * 4,
    )
    return k(xp, pred)


def _segsum_body(
    table_hbm, src2_hbm, dst2_hbm, out_hbm,
    si, di, r0, r1, r2, r3, zbuf, acc, isem_s, isem_d, gsem, ssem,
):
    cid = lax.axis_index("c")
    sid = lax.axis_index("s")
    nc = jnp.where(cid == 0, NC_SEG[0], NC_SEG[1])
    wbase = jnp.where(cid == 0, sid * NC_SEG[0], 16 * NC_SEG[0] + sid * NC_SEG[1])
    rows = [r0, r1, r2, r3]

    @pl.loop(0, 64)
    def _(r):
        zbuf[r, pl.ds(0, 16)] = jnp.zeros((16,), _f32)
        zbuf[r, pl.ds(16, 16)] = jnp.zeros((16,), _f32)

    @pl.loop(0, ROWS_PER_TILE // 64)
    def _(j):
        pltpu.async_copy(zbuf, acc.at[pl.ds(sid * ROWS_PER_TILE + j * 64, 64)], ssem.at[0])

    @pl.loop(0, ROWS_PER_TILE // 64)
    def _(j):
        pltpu.make_async_copy(zbuf, acc.at[pl.ds(0, 64)], ssem.at[0]).wait()

    plsc.subcore_barrier()

    def fire_idx(c, m8):
        pltpu.async_copy(src2_hbm.at[wbase + c], si.at[m8], isem_s.at[m8])
        pltpu.async_copy(dst2_hbm.at[wbase + c], di.at[m8], isem_d.at[m8])

    def wait_idx(m8):
        pltpu.make_async_copy(src2_hbm.at[0], si.at[m8], isem_s.at[m8]).wait()
        pltpu.make_async_copy(dst2_hbm.at[0], di.at[m8], isem_d.at[m8]).wait()

    def fire_gather(m8, m4):
        pltpu.async_copy(table_hbm.at[si.at[m8]], rows[m4], gsem.at[m4])

    def wait_gather(m4):
        pltpu.make_async_copy(table_hbm.at[pl.ds(0, CHUNK)], rows[m4], gsem.at[m4]).wait()

    def fire_scatter(m8, m4):
        pltpu.async_copy(rows[m4], acc.at[di.at[m8]], ssem.at[m4], add=True)

    def wait_scatter(m4):
        pltpu.make_async_copy(rows[m4], acc.at[pl.ds(0, CHUNK)], ssem.at[m4]).wait()

    fire_idx(0, 0)
    fire_idx(1, 1)
    fire_idx(2, 2)
    wait_idx(0)
    fire_gather(0, 0)
    wait_idx(1)
    fire_gather(1, 1)

    @pl.loop(0, nc, step=8)
    def _(cc):
        for b in range(8):
            c = cc + b

            @pl.when(c + 3 < nc)
            def _():
                fire_idx(c + 3, (b + 3) % 8)

            wait_gather(b % 4)
            fire_scatter(b, b % 4)

            @pl.when(c + 2 < nc)
            def _():
                wait_idx((b + 2) % 8)

                @pl.when(c >= 2)
                def _():
                    wait_scatter((b + 2) % 4)

                fire_gather((b + 2) % 8, (b + 2) % 4)

    for b in range(4):
        wait_scatter(b)

    plsc.subcore_barrier()
    pltpu.sync_copy(
        acc.at[pl.ds(sid * ROWS_PER_TILE, ROWS_PER_TILE)],
        out_hbm.at[cid, pl.ds(sid * ROWS_PER_TILE, ROWS_PER_TILE)],
    )


def _segsum(table, src2, dst2):
    k = pl.kernel(
        _segsum_body,
        out_type=jax.ShapeDtypeStruct((2, NP, 32), _f32),
        mesh=_vmesh(),
        compiler_params=_SC_PARAMS,
        scratch_types=[
            pltpu.VMEM((8, CHUNK), _i32),
            pltpu.VMEM((8, CHUNK), _i32),
            pltpu.VMEM((CHUNK, 32), _f32),
            pltpu.VMEM((CHUNK, 32), _f32),
            pltpu.VMEM((CHUNK, 32), _f32),
            pltpu.VMEM((CHUNK, 32), _f32),
            pltpu.VMEM((64, 32), _f32),
            pltpu.VMEM_SHARED((NP, 32), _f32),
            pltpu.SemaphoreType.DMA((8,)),
            pltpu.SemaphoreType.DMA((8,)),
            pltpu.SemaphoreType.DMA((4,)),
            pltpu.SemaphoreType.DMA((4,)),
        ],
    )
    return k(table, src2, dst2)


def _edge_body(
    aw_hbm, bw_hbm, src_hbm, dst_hbm, wm2_hbm, bm2_hbm, o0_hbm, o1_hbm,
    sidx_all, didx_all, ga0, ga1, gb0, gb1, ov00, ov01, ov10, ov11, wm2_v, bm2_v, wrot,
    ga_s0, ga_s1, gb_s0, gb_s1, o0_s0, o0_s1, o1_s0, o1_s1,
):
    cid = lax.axis_index("c")
    sid = lax.axis_index("s")
    nc = jnp.where(cid == 0, NC_EDGE[0], NC_EDGE[1])
    wstart = jnp.where(cid == 0, sid * NC_EDGE[0], 16 * NC_EDGE[0] + sid * NC_EDGE[1])
    ebase = wstart * CHUNK
    iot = lax.iota(_i32, 16)
    pltpu.sync_copy(wm2_hbm, wm2_v)
    pltpu.sync_copy(bm2_hbm, bm2_v)
    bv = bm2_v[...]

    @pl.loop(0, 32)
    def _(k):
        kl = jnp.bitwise_and(k + iot, 31)
        z16 = jnp.zeros((16,), _i32)
        o16 = jnp.ones((16,), _i32)
        wrot[0, k] = plsc.load_gather(wm2_v, [z16, kl]) * (1.0 / QS)
        wrot[1, k] = plsc.load_gather(wm2_v, [z16, kl + 32]) * (1.0 / QS)
        wrot[2, k] = plsc.load_gather(wm2_v, [o16, kl]) * (1.0 / QS)
        wrot[3, k] = plsc.load_gather(wm2_v, [o16, kl + 32]) * (1.0 / QS)

    pltpu.sync_copy(src_hbm.at[pl.ds(ebase, IDX_PAD)], sidx_all)
    pltpu.sync_copy(dst_hbm.at[pl.ds(ebase, IDX_PAD)], didx_all)
    ga = [ga0, ga1]
    gb = [gb0, gb1]
    ov0 = [ov00, ov01]
    ov1 = [ov10, ov11]
    gasem = [ga_s0, ga_s1]
    gbsem = [gb_s0, gb_s1]
    o0sem = [o0_s0, o0_s1]
    o1sem = [o1_s0, o1_s1]
    rows_g = [g * 16 + iot for g in range(8)]

    def fire_gather(c, p):
        pltpu.async_copy(
            aw_hbm.at[sidx_all.at[pl.ds(c * CHUNK, CHUNK)]], ga[p], gasem[p]
        )
        pltpu.async_copy(
            bw_hbm.at[didx_all.at[pl.ds(c * CHUNK, CHUNK)]], gb[p], gbsem[p]
        )

    def wait_gather(p):
        pltpu.make_async_copy(aw_hbm.at[pl.ds(0, CHUNK)], ga[p], gasem[p]).wait()
        pltpu.make_async_copy(bw_hbm.at[pl.ds(0, CHUNK)], gb[p], gbsem[p]).wait()

    def wait_out(p):
        pltpu.make_async_copy(ov0[p], o0_hbm.at[pl.ds(0, CHUNK)], o0sem[p]).wait()
        pltpu.make_async_copy(ov1[p], o1_hbm.at[pl.ds(0, CHUNK)], o1sem[p]).wait()

    fire_gather(0, 0)
    fire_gather(1, 1)

    @pl.loop(0, nc, step=2)
    def _(cc):
        for b in range(2):
            c = cc + b
            p = b
            wait_gather(p)

            @pl.when(c >= 2)
            def _():
                wait_out(p)

            init = tuple(jnp.zeros((16,), _f32) + bv[0] for _ in range(8)) + tuple(
                jnp.zeros((16,), _f32) + bv[1] for _ in range(8)
            )

            @pl.loop(0, 32, init_carry=init)
            def accs(k, carry):
                kl = jnp.bitwise_and(k + iot, 31)
                w0h = wrot[0, k]
                w0l = wrot[1, k]
                w1h = wrot[2, k]
                w1l = wrot[3, k]
                out = []
                out1 = []
                for g in range(8):
                    wa = plsc.load_gather(ga[p], [rows_g[g], kl])
                    wb = plsc.load_gather(gb[p], [rows_g[g], kl])
                    sh = lax.shift_right_arithmetic(wa, 16) + lax.shift_right_arithmetic(wb, 16)
                    sl = lax.shift_right_arithmetic(
                        lax.shift_left(wa, 16), 16
                    ) + lax.shift_right_arithmetic(lax.shift_left(wb, 16), 16)
                    rh = jnp.maximum(sh, 0).astype(_f32)
                    rl = jnp.maximum(sl, 0).astype(_f32)
                    out.append(carry[g] + rh * w0h + rl * w0l)
                    out1.append(carry[8 + g] + rh * w1h + rl * w1l)
                return tuple(out) + tuple(out1)

            for g in range(8):
                ov0[p][pl.ds(g * 16, 16)] = accs[g]
                ov1[p][pl.ds(g * 16, 16)] = accs[8 + g]

            base = ebase + c * CHUNK
            pltpu.async_copy(ov0[p], o0_hbm.at[pl.ds(base, CHUNK)], o0sem[p])
            pltpu.async_copy(ov1[p], o1_hbm.at[pl.ds(base, CHUNK)], o1sem[p])

            @pl.when(c + 2 < nc)
            def _():
                fire_gather(c + 2, p)

    for b in range(2):
        wait_out(b)


def _edge_mlp(aw, bw, srcp, dstp, wm2, bm2):
    k = pl.kernel(
        _edge_body,
        out_type=(
            jax.ShapeDtypeStruct((EP,), _f32),
            jax.ShapeDtypeStruct((EP,), _f32),
        ),
        mesh=_vmesh(),
        compiler_params=_SC_PARAMS,
        scratch_types=[
            pltpu.VMEM((IDX_PAD,), _i32),
            pltpu.VMEM((IDX_PAD,), _i32),
            pltpu.VMEM((CHUNK, 32), _i32),
            pltpu.VMEM((CHUNK, 32), _i32),
            pltpu.VMEM((CHUNK, 32), _i32),
            pltpu.VMEM((CHUNK, 32), _i32),
            pltpu.VMEM((CHUNK,), _f32),
            pltpu.VMEM((CHUNK,), _f32),
            pltpu.VMEM((CHUNK,), _f32),
            pltpu.VMEM((CHUNK,), _f32),
            pltpu.VMEM((2, 64), _f32),
            pltpu.VMEM((16,), _f32),
            pltpu.VMEM((4, 32, 16), _f32),
        ] + [pltpu.SemaphoreType.DMA] * 8,
    )
    return k(aw, bw, srcp, dstp, wm2, bm2)


_BLK = 1024
_GRID = NP // _BLK


def _dot(a, b):
    return lax.dot_general(a, b, (((1,), (0,)), ((), ())))


def _tc1_body(acc_ref, h0_ref, w1l_ref, w1r_ref, b1_ref, h1a_ref, h1b_ref, rcp_ref):
    acc = acc_ref[0] + acc_ref[1]
    cnt = jnp.maximum(acc[:, 25:26], 1.0)
    mean = acc / cnt
    out = _dot(mean, w1l_ref[...]) + b1_ref[...] + _dot(h0_ref[...], w1r_ref[...])
    norm = jnp.sqrt(jnp.sum(out * out, axis=1, keepdims=True))
    h = jnp.maximum(out / jnp.maximum(norm, 1e-12), 0.0)
    h1a_ref[...] = h[:, :32]
    h1b_ref[...] = h[:, 32:]
    rcp_ref[...] = jnp.broadcast_to(cnt, (_BLK, 8))


def _tc1(acc, h0, w1lT, w1rT, b1):
    return pl.pallas_call(
        _tc1_body,
        grid=(_GRID,),
        in_specs=[
            pl.BlockSpec((2, _BLK, 32), lambda i: (0, i, 0)),
            pl.BlockSpec((_BLK, 32), lambda i: (i, 0)),
            pl.BlockSpec((32, 64), lambda i: (0, 0)),
            pl.BlockSpec((32, 64), lambda i: (0, 0)),
            pl.BlockSpec((1, 64), lambda i: (0, 0)),
        ],
        out_specs=[
            pl.BlockSpec((_BLK, 32), lambda i: (i, 0)),
            pl.BlockSpec((_BLK, 32), lambda i: (i, 0)),
            pl.BlockSpec((_BLK, 8), lambda i: (i, 0)),
        ],
        out_shape=[
            jax.ShapeDtypeStruct((NP, 32), _f32),
            jax.ShapeDtypeStruct((NP, 32), _f32),
            jax.ShapeDtypeStruct((NP, 8), _f32),
        ],
    )(acc, h0, w1lT, w1rT, b1)


def _tc2_body(
    acca_ref, accb_ref, h1a_ref, h1b_ref, rcp_ref,
    w2l_ref, w2r_ref, b2_ref, wm1s_ref, wm1d_ref, bm1_ref,
    aw_ref, bw_ref,
):
    cnt = rcp_ref[:, 0:1]
    mean = jnp.concatenate(
        [(acca_ref[0] + acca_ref[1]) / cnt, (accb_ref[0] + accb_ref[1]) / cnt], axis=1
    )
    h1 = jnp.concatenate([h1a_ref[...], h1b_ref[...]], axis=1)
    out = _dot(mean, w2l_ref[...]) + b2_ref[...] + _dot(h1, w2r_ref[...])
    norm = jnp.sqrt(jnp.sum(out * out, axis=1, keepdims=True))
    h2 = jnp.maximum(out / jnp.maximum(norm, 1e-12), 0.0)
    af = _dot(h2, wm1s_ref[...])
    bf = _dot(h2, wm1d_ref[...]) + bm1_ref[...]

    def pack(x):
        q = lax.convert_element_type(
            lax.round(jnp.clip(x, -1.6, 1.6) * QS), jnp.int32
        )
        return (q[:, :32] << 16) | (q[:, 32:] & 0xFFFF)

    aw_ref[...] = pack(af)
    bw_ref[...] = pack(bf)


def _tc2(acca, accb, h1a, h1b, rcp8, w2lT, w2rT, b2, wm1sT, wm1dT, bm1):
    return pl.pallas_call(
        _tc2_body,
        grid=(_GRID,),
        in_specs=[
            pl.BlockSpec((2, _BLK, 32), lambda i: (0, i, 0)),
            pl.BlockSpec((2, _BLK, 32), lambda i: (0, i, 0)),
            pl.BlockSpec((_BLK, 32), lambda i: (i, 0)),
            pl.BlockSpec((_BLK, 32), lambda i: (i, 0)),
            pl.BlockSpec((_BLK, 8), lambda i: (i, 0)),
            pl.BlockSpec((64, 64), lambda i: (0, 0)),
            pl.BlockSpec((64, 64), lambda i: (0, 0)),
            pl.BlockSpec((1, 64), lambda i: (0, 0)),
            pl.BlockSpec((64, 64), lambda i: (0, 0)),
            pl.BlockSpec((64, 64), lambda i: (0, 0)),
            pl.BlockSpec((1, 64), lambda i: (0, 0)),
        ],
        out_specs=[
            pl.BlockSpec((_BLK, 32), lambda i: (i, 0)),
            pl.BlockSpec((_BLK, 32), lambda i: (i, 0)),
        ],
        out_shape=[
            jax.ShapeDtypeStruct((NP, 32), jnp.int32),
            jax.ShapeDtypeStruct((NP, 32), jnp.int32),
        ],
    )(acca, accb, h1a, h1b, rcp8, w2lT, w2rT, b2, wm1sT, wm1dT, bm1)


def kernel(x, edge_index, pred_embed, W1l, b1l, W1r, W2l, b2l, W2r, Wm1, bm1, Wm2, bm2):
    xp = jnp.zeros((NP, 16), _f32).at[:N, :10].set(x)
    pad = jnp.full((EPI - E,), N, _i32)
    srcp = jnp.concatenate([edge_index[0], pad])
    dstp = jnp.concatenate([edge_index[1], pad])

    w1lT = jnp.zeros((32, 64), _f32).at[:25].set(W1l.T)
    w1rT = jnp.zeros((32, 64), _f32).at[:25].set(W1r.T)

    src2 = srcp.reshape(EPI // CHUNK, CHUNK)
    dst2 = dstp.reshape(EPI // CHUNK, CHUNK)

    h0 = _build_h0(xp, pred_embed)
    acc1 = _segsum(h0, src2, dst2)
    h1a, h1b, rcp8 = _tc1(acc1, h0, w1lT, w1rT, b1l.reshape(1, 64))
    acca = _segsum(h1a, src2, dst2)
    accb = _segsum(h1b, src2, dst2)
    aw, bw = _tc2(
        acca, accb, h1a, h1b, rcp8,
        W2l.T, W2r.T, b2l.reshape(1, 64),
        Wm1[:, :64].T, Wm1[:, 64:].T, bm1.reshape(1, 64),
    )
    o0, o1 = _edge_mlp(
        aw, bw, srcp, dstp, Wm2, jnp.zeros((16,), _f32).at[:2].set(bm2)
    )
    return jnp.stack([o0[:E], o1[:E]], axis=1)


# speedup vs baseline: 7.2096x; 1.0523x over previous
import functools

import jax
import jax.numpy as jnp
from jax import lax
from jax.experimental import pallas as pl
from jax.experimental.pallas import tpu as pltpu
from jax.experimental.pallas import tpu_sc as plsc

N = 50000
E = 800000
NP = 53248
EP = 819200
CHUNK = 128
NWORK = 32
EDGES_PER_W = EP // NWORK
NCHUNK_E = EDGES_PER_W // CHUNK
NC_EDGE = (240, 160)
NC_SEG = (352, 48)
NCMAX_EDGE = max(NC_EDGE)
QS = 20000.0
IDX_PAD = NCMAX_EDGE * CHUNK
EPI = (16 * NC_EDGE[0] + 15 * NC_EDGE[1]) * CHUNK + IDX_PAD
NODE_CHUNKS = NP // CHUNK
ROWS_PER_TILE = NP // 16

_f32 = jnp.float32
_i32 = jnp.int32


def _vmesh():
    return plsc.VectorSubcoreMesh(
        core_axis_name="c", subcore_axis_name="s", num_cores=2, num_subcores=16
    )


_SC_PARAMS = pltpu.CompilerParams(
    needs_layout_passes=False, use_tc_tiling_on_sc=False
)


def _worker_id():
    return lax.axis_index("c") * 16 + lax.axis_index("s")


def _build_h0_body(
    xp_hbm, pred_hbm, h0_hbm, xv0, xv1, pred_v, hv0, hv1, xs0, xs1, hs0, hs1
):
    w = _worker_id()
    iot = lax.iota(_i32, 16)
    xv = [xv0, xv1]
    hv = [hv0, hv1]
    xsem = [xs0, xs1]
    hsem = [hs0, hs1]
    cpw = NODE_CHUNKS // NWORK
    c0 = w * cpw
    pltpu.sync_copy(pred_hbm, pred_v)

    def fire_x(i, p):
        pltpu.async_copy(xp_hbm.at[pl.ds((c0 + i) * CHUNK, CHUNK)], xv[p], xsem[p])

    def wait_x(p):
        pltpu.make_async_copy(xp_hbm.at[pl.ds(0, CHUNK)], xv[p], xsem[p]).wait()

    def wait_h(p):
        pltpu.make_async_copy(hv[p], h0_hbm.at[pl.ds(0, CHUNK)], hsem[p]).wait()

    def stage(i, p, first, last):
        wait_x(p)
        if not last:
            fire_x(i + 1, 1 - p)
        if not first:
            wait_h(p)
        for g in range(8):
            rows = g * 16 + iot
            pid = plsc.load_gather(xv[p], [rows, jnp.full((16,), 1, _i32)]).astype(_i32)
            v = plsc.load_gather(xv[p], [rows, jnp.full((16,), 0, _i32)])
            plsc.store_scatter(hv[p], [rows, jnp.full((16,), 0, _i32)], v)
            for j in range(8):
                v = plsc.load_gather(xv[p], [rows, jnp.full((16,), 2 + j, _i32)])
                plsc.store_scatter(hv[p], [rows, jnp.full((16,), 1 + j, _i32)], v)
            for j in range(16):
                v = plsc.load_gather(pred_v, [pid, jnp.full((16,), j, _i32)])
                plsc.store_scatter(hv[p], [rows, jnp.full((16,), 9 + j, _i32)], v)
            plsc.store_scatter(
                hv[p], [rows, jnp.full((16,), 25, _i32)], jnp.ones((16,), _f32)
            )
            for j in range(26, 32):
                plsc.store_scatter(
                    hv[p], [rows, jnp.full((16,), j, _i32)], jnp.zeros((16,), _f32)
                )
        pltpu.async_copy(hv[p], h0_hbm.at[pl.ds((c0 + i) * CHUNK, CHUNK)], hsem[p])

    fire_x(0, 0)
    stage(0, 0, first=True, last=False)
    stage(1, 1, first=True, last=False)

    @pl.loop(2, cpw - 1, step=2)
    def _(ii):
        for b in range(2):
            stage(ii + b, b, first=False, last=False)

    stage(cpw - 1, 0, first=False, last=True)
    wait_h(0)
    wait_h(1)


def _build_h0(xp, pred):
    k = pl.kernel(
        _build_h0_body,
        out_type=jax.ShapeDtypeStruct((NP, 32), _f32),
        mesh=_vmesh(),
        compiler_params=_SC_PARAMS,
        scratch_types=[
            pltpu.VMEM((CHUNK, 16), _f32),
            pltpu.VMEM((CHUNK, 16), _f32),
            pltpu.VMEM((1001, 16), _f32),
            pltpu.VMEM((CHUNK, 32), _f32),
            pltpu.VMEM((CHUNK, 32), _f32),
        ] + [pltpu.SemaphoreType.DMA] * 4,
    )
    return k(xp, pred)


def _segsum_body(
    table_hbm, src2_hbm, dst2_hbm, out_hbm,
    si, di, r0, r1, r2, r3, zbuf, acc, isem_s, isem_d, gsem, ssem,
):
    cid = lax.axis_index("c")
    sid = lax.axis_index("s")
    nc = jnp.where(cid == 0, NC_SEG[0], NC_SEG[1])
    wbase = jnp.where(cid == 0, sid * NC_SEG[0], 16 * NC_SEG[0] + sid * NC_SEG[1])
    rows = [r0, r1, r2, r3]

    @pl.loop(0, 64)
    def _(r):
        zbuf[r, pl.ds(0, 16)] = jnp.zeros((16,), _f32)
        zbuf[r, pl.ds(16, 16)] = jnp.zeros((16,), _f32)

    @pl.loop(0, ROWS_PER_TILE // 64)
    def _(j):
        pltpu.async_copy(zbuf, acc.at[pl.ds(sid * ROWS_PER_TILE + j * 64, 64)], ssem.at[0])

    @pl.loop(0, ROWS_PER_TILE // 64)
    def _(j):
        pltpu.make_async_copy(zbuf, acc.at[pl.ds(0, 64)], ssem.at[0]).wait()

    plsc.subcore_barrier()

    def fire_idx(c, m8):
        pltpu.async_copy(src2_hbm.at[wbase + c], si.at[m8], isem_s.at[m8])
        pltpu.async_copy(dst2_hbm.at[wbase + c], di.at[m8], isem_d.at[m8])

    def wait_idx(m8):
        pltpu.make_async_copy(src2_hbm.at[0], si.at[m8], isem_s.at[m8]).wait()
        pltpu.make_async_copy(dst2_hbm.at[0], di.at[m8], isem_d.at[m8]).wait()

    def fire_gather(m8, m4):
        pltpu.async_copy(table_hbm.at[si.at[m8]], rows[m4], gsem.at[m4])

    def wait_gather(m4):
        pltpu.make_async_copy(table_hbm.at[pl.ds(0, CHUNK)], rows[m4], gsem.at[m4]).wait()

    def fire_scatter(m8, m4):
        pltpu.async_copy(rows[m4], acc.at[di.at[m8]], ssem.at[m4], add=True)

    def wait_scatter(m4):
        pltpu.make_async_copy(rows[m4], acc.at[pl.ds(0, CHUNK)], ssem.at[m4]).wait()

    fire_idx(0, 0)
    fire_idx(1, 1)
    fire_idx(2, 2)
    wait_idx(0)
    fire_gather(0, 0)
    wait_idx(1)
    fire_gather(1, 1)

    @pl.loop(0, nc, step=8)
    def _(cc):
        for b in range(8):
            c = cc + b

            @pl.when(c + 3 < nc)
            def _():
                fire_idx(c + 3, (b + 3) % 8)

            wait_gather(b % 4)
            fire_scatter(b, b % 4)

            @pl.when(c + 2 < nc)
            def _():
                wait_idx((b + 2) % 8)

                @pl.when(c >= 2)
                def _():
                    wait_scatter((b + 2) % 4)

                fire_gather((b + 2) % 8, (b + 2) % 4)

    for b in range(4):
        wait_scatter(b)

    plsc.subcore_barrier()
    pltpu.sync_copy(
        acc.at[pl.ds(sid * ROWS_PER_TILE, ROWS_PER_TILE)],
        out_hbm.at[cid, pl.ds(sid * ROWS_PER_TILE, ROWS_PER_TILE)],
    )


def _segsum(table, src2, dst2):
    k = pl.kernel(
        _segsum_body,
        out_type=jax.ShapeDtypeStruct((2, NP, 32), _f32),
        mesh=_vmesh(),
        compiler_params=_SC_PARAMS,
        scratch_types=[
            pltpu.VMEM((8, CHUNK), _i32),
            pltpu.VMEM((8, CHUNK), _i32),
            pltpu.VMEM((CHUNK, 32), _f32),
            pltpu.VMEM((CHUNK, 32), _f32),
            pltpu.VMEM((CHUNK, 32), _f32),
            pltpu.VMEM((CHUNK, 32), _f32),
            pltpu.VMEM((64, 32), _f32),
            pltpu.VMEM_SHARED((NP, 32), _f32),
            pltpu.SemaphoreType.DMA((8,)),
            pltpu.SemaphoreType.DMA((8,)),
            pltpu.SemaphoreType.DMA((4,)),
            pltpu.SemaphoreType.DMA((4,)),
        ],
    )
    return k(table, src2, dst2)


def _edge_body(
    aw_hbm, bw_hbm, src_hbm, dst_hbm, wm2_hbm, bm2_hbm, o0_hbm, o1_hbm,
    sidx_all, didx_all, ga0, ga1, gb0, gb1, ov00, ov01, ov10, ov11, wm2_v, bm2_v, wrot,
    ga_s0, ga_s1, gb_s0, gb_s1, o0_s0, o0_s1, o1_s0, o1_s1,
):
    cid = lax.axis_index("c")
    sid = lax.axis_index("s")
    nc = jnp.where(cid == 0, NC_EDGE[0], NC_EDGE[1])
    wstart = jnp.where(cid == 0, sid * NC_EDGE[0], 16 * NC_EDGE[0] + sid * NC_EDGE[1])
    ebase = wstart * CHUNK
    iot = lax.iota(_i32, 16)
    pltpu.sync_copy(wm2_hbm, wm2_v)
    pltpu.sync_copy(bm2_hbm, bm2_v)
    bv = bm2_v[...]

    @pl.loop(0, 32)
    def _(k):
        kl = jnp.bitwise_and(k + iot, 31)
        z16 = jnp.zeros((16,), _i32)
        o16 = jnp.ones((16,), _i32)
        wrot[0, k] = plsc.load_gather(wm2_v, [z16, kl]) * (1.0 / QS)
        wrot[1, k] = plsc.load_gather(wm2_v, [z16, kl + 32]) * (1.0 / QS)
        wrot[2, k] = plsc.load_gather(wm2_v, [o16, kl]) * (1.0 / QS)
        wrot[3, k] = plsc.load_gather(wm2_v, [o16, kl + 32]) * (1.0 / QS)

    pltpu.sync_copy(src_hbm.at[pl.ds(ebase, IDX_PAD)], sidx_all)
    pltpu.sync_copy(dst_hbm.at[pl.ds(ebase, IDX_PAD)], didx_all)
    ga = [ga0, ga1]
    gb = [gb0, gb1]
    ov0 = [ov00, ov01]
    ov1 = [ov10, ov11]
    gasem = [ga_s0, ga_s1]
    gbsem = [gb_s0, gb_s1]
    o0sem = [o0_s0, o0_s1]
    o1sem = [o1_s0, o1_s1]
    rows_g = [g * 16 + iot for g in range(8)]

    def fire_gather(c, p):
        pltpu.async_copy(
            aw_hbm.at[sidx_all.at[pl.ds(c * CHUNK, CHUNK)]], ga[p], gasem[p]
        )
        pltpu.async_copy(
            bw_hbm.at[didx_all.at[pl.ds(c * CHUNK, CHUNK)]], gb[p], gbsem[p]
        )

    def wait_gather(p):
        pltpu.make_async_copy(aw_hbm.at[pl.ds(0, CHUNK)], ga[p], gasem[p]).wait()
        pltpu.make_async_copy(bw_hbm.at[pl.ds(0, CHUNK)], gb[p], gbsem[p]).wait()

    def wait_out(p):
        pltpu.make_async_copy(ov0[p], o0_hbm.at[pl.ds(0, CHUNK)], o0sem[p]).wait()
        pltpu.make_async_copy(ov1[p], o1_hbm.at[pl.ds(0, CHUNK)], o1sem[p]).wait()

    fire_gather(0, 0)
    fire_gather(1, 1)

    @pl.loop(0, nc, step=2)
    def _(cc):
        for b in range(2):
            c = cc + b
            p = b
            wait_gather(p)

            @pl.when(c >= 2)
            def _():
                wait_out(p)

            init = tuple(jnp.zeros((16,), _f32) + bv[0] for _ in range(8)) + tuple(
                jnp.zeros((16,), _f32) + bv[1] for _ in range(8)
            )

            @pl.loop(0, 32, init_carry=init)
            def accs(k, carry):
                kl = jnp.bitwise_and(k + iot, 31)
                w0h = wrot[0, k]
                w0l = wrot[1, k]
                w1h = wrot[2, k]
                w1l = wrot[3, k]
                out = []
                out1 = []
                for g in range(8):
                    wa = plsc.load_gather(ga[p], [rows_g[g], kl])
                    wb = plsc.load_gather(gb[p], [rows_g[g], kl])
                    sh = lax.shift_right_arithmetic(wa, 16) + lax.shift_right_arithmetic(wb, 16)
                    sl = lax.shift_right_arithmetic(
                        lax.shift_left(wa, 16), 16
                    ) + lax.shift_right_arithmetic(lax.shift_left(wb, 16), 16)
                    rh = jnp.maximum(sh, 0).astype(_f32)
                    rl = jnp.maximum(sl, 0).astype(_f32)
                    out.append(carry[g] + rh * w0h + rl * w0l)
                    out1.append(carry[8 + g] + rh * w1h + rl * w1l)
                return tuple(out) + tuple(out1)

            for g in range(8):
                ov0[p][pl.ds(g * 16, 16)] = accs[g]
                ov1[p][pl.ds(g * 16, 16)] = accs[8 + g]

            base = ebase + c * CHUNK
            pltpu.async_copy(ov0[p], o0_hbm.at[pl.ds(base, CHUNK)], o0sem[p])
            pltpu.async_copy(ov1[p], o1_hbm.at[pl.ds(base, CHUNK)], o1sem[p])

            @pl.when(c + 2 < nc)
            def _():
                fire_gather(c + 2, p)

    for b in range(2):
        wait_out(b)


def _edge_mlp(aw, bw, srcp, dstp, wm2, bm2):
    k = pl.kernel(
        _edge_body,
        out_type=(
            jax.ShapeDtypeStruct((EP,), _f32),
            jax.ShapeDtypeStruct((EP,), _f32),
        ),
        mesh=_vmesh(),
        compiler_params=_SC_PARAMS,
        scratch_types=[
            pltpu.VMEM((IDX_PAD,), _i32),
            pltpu.VMEM((IDX_PAD,), _i32),
            pltpu.VMEM((CHUNK, 32), _i32),
            pltpu.VMEM((CHUNK, 32), _i32),
            pltpu.VMEM((CHUNK, 32), _i32),
            pltpu.VMEM((CHUNK, 32), _i32),
            pltpu.VMEM((CHUNK,), _f32),
            pltpu.VMEM((CHUNK,), _f32),
            pltpu.VMEM((CHUNK,), _f32),
            pltpu.VMEM((CHUNK,), _f32),
            pltpu.VMEM((2, 64), _f32),
            pltpu.VMEM((16,), _f32),
            pltpu.VMEM((4, 32, 16), _f32),
        ] + [pltpu.SemaphoreType.DMA] * 8,
    )
    return k(aw, bw, srcp, dstp, wm2, bm2)


_BLK = 1024
_GRID = NP // _BLK


def _dot(a, b):
    return lax.dot_general(a, b, (((1,), (0,)), ((), ())))


def _tc1_body(acc_ref, h0_ref, w1l_ref, w1r_ref, b1_ref, h1a_ref, h1b_ref, rcp_ref):
    acc = acc_ref[0] + acc_ref[1]
    cnt = jnp.maximum(acc[:, 25:26], 1.0)
    mean = acc / cnt
    out = _dot(mean, w1l_ref[...]) + b1_ref[...] + _dot(h0_ref[...], w1r_ref[...])
    norm = jnp.sqrt(jnp.sum(out * out, axis=1, keepdims=True))
    h = jnp.maximum(out / jnp.maximum(norm, 1e-12), 0.0)
    h1a_ref[...] = h[:, :32]
    h1b_ref[...] = h[:, 32:]
    rcp_ref[...] = jnp.broadcast_to(cnt, (_BLK, 8))


def _tc1(acc, h0, w1lT, w1rT, b1):
    return pl.pallas_call(
        _tc1_body,
        grid=(_GRID,),
        in_specs=[
            pl.BlockSpec((2, _BLK, 32), lambda i: (0, i, 0)),
            pl.BlockSpec((_BLK, 32), lambda i: (i, 0)),
            pl.BlockSpec((32, 64), lambda i: (0, 0)),
            pl.BlockSpec((32, 64), lambda i: (0, 0)),
            pl.BlockSpec((1, 64), lambda i: (0, 0)),
        ],
        out_specs=[
            pl.BlockSpec((_BLK, 32), lambda i: (i, 0)),
            pl.BlockSpec((_BLK, 32), lambda i: (i, 0)),
            pl.BlockSpec((_BLK, 8), lambda i: (i, 0)),
        ],
        out_shape=[
            jax.ShapeDtypeStruct((NP, 32), _f32),
            jax.ShapeDtypeStruct((NP, 32), _f32),
            jax.ShapeDtypeStruct((NP, 8), _f32),
        ],
    )(acc, h0, w1lT, w1rT, b1)


def _tc2_body(
    acca_ref, accb_ref, h1a_ref, h1b_ref, rcp_ref,
    w2l_ref, w2r_ref, b2_ref, wm1s_ref, wm1d_ref, bm1_ref,
    aw_ref, bw_ref,
):
    cnt = rcp_ref[:, 0:1]
    mean = jnp.concatenate(
        [(acca_ref[0] + acca_ref[1]) / cnt, (accb_ref[0] + accb_ref[1]) / cnt], axis=1
    )
    h1 = jnp.concatenate([h1a_ref[...], h1b_ref[...]], axis=1)
    out = _dot(mean, w2l_ref[...]) + b2_ref[...] + _dot(h1, w2r_ref[...])
    norm = jnp.sqrt(jnp.sum(out * out, axis=1, keepdims=True))
    h2 = jnp.maximum(out / jnp.maximum(norm, 1e-12), 0.0)
    af = _dot(h2, wm1s_ref[...])
    bf = _dot(h2, wm1d_ref[...]) + bm1_ref[...]

    def pack(x):
        q = lax.convert_element_type(
            lax.round(jnp.clip(x, -1.6, 1.6) * QS), jnp.int32
        )
        return (q[:, :32] << 16) | (q[:, 32:] & 0xFFFF)

    aw_ref[...] = pack(af)
    bw_ref[...] = pack(bf)


def _tc2(acca, accb, h1a, h1b, rcp8, w2lT, w2rT, b2, wm1sT, wm1dT, bm1):
    return pl.pallas_call(
        _tc2_body,
        grid=(_GRID,),
        in_specs=[
            pl.BlockSpec((2, _BLK, 32), lambda i: (0, i, 0)),
            pl.BlockSpec((2, _BLK, 32), lambda i: (0, i, 0)),
            pl.BlockSpec((_BLK, 32), lambda i: (i, 0)),
            pl.BlockSpec((_BLK, 32), lambda i: (i, 0)),
            pl.BlockSpec((_BLK, 8), lambda i: (i, 0)),
            pl.BlockSpec((64, 64), lambda i: (0, 0)),
            pl.BlockSpec((64, 64), lambda i: (0, 0)),
            pl.BlockSpec((1, 64), lambda i: (0, 0)),
            pl.BlockSpec((64, 64), lambda i: (0, 0)),
            pl.BlockSpec((64, 64), lambda i: (0, 0)),
            pl.BlockSpec((1, 64), lambda i: (0, 0)),
        ],
        out_specs=[
            pl.BlockSpec((_BLK, 32), lambda i: (i, 0)),
            pl.BlockSpec((_BLK, 32), lambda i: (i, 0)),
        ],
        out_shape=[
            jax.ShapeDtypeStruct((NP, 32), jnp.int32),
            jax.ShapeDtypeStruct((NP, 32), jnp.int32),
        ],
    )(acca, accb, h1a, h1b, rcp8, w2lT, w2rT, b2, wm1sT, wm1dT, bm1)


def kernel(x, edge_index, pred_embed, W1l, b1l, W1r, W2l, b2l, W2r, Wm1, bm1, Wm2, bm2):
    xp = jnp.zeros((NP, 16), _f32).at[:N, :10].set(x)
    pad = jnp.full((EPI - E,), N, _i32)
    srcp = jnp.concatenate([edge_index[0], pad])
    dstp = jnp.concatenate([edge_index[1], pad])

    w1lT = jnp.zeros((32, 64), _f32).at[:25].set(W1l.T)
    w1rT = jnp.zeros((32, 64), _f32).at[:25].set(W1r.T)

    src2 = srcp.reshape(EPI // CHUNK, CHUNK)
    dst2 = dstp.reshape(EPI // CHUNK, CHUNK)

    h0 = _build_h0(xp, pred_embed)
    acc1 = _segsum(h0, src2, dst2)
    h1a, h1b, rcp8 = _tc1(acc1, h0, w1lT, w1rT, b1l.reshape(1, 64))
    acca = _segsum(h1a, src2, dst2)
    accb = _segsum(h1b, src2, dst2)
    aw, bw = _tc2(
        acca, accb, h1a, h1b, rcp8,
        W2l.T, W2r.T, b2l.reshape(1, 64),
        Wm1[:, :64].T, Wm1[:, 64:].T, bm1.reshape(1, 64),
    )
    o0, o1 = _edge_mlp(
        aw, bw, srcp, dstp, Wm2, jnp.zeros((16,), _f32).at[:2].set(bm2)
    )
    return jnp.stack([o0[:E], o1[:E]], axis=1)
